# Initial kernel scaffold; baseline (speedup 1.0000x reference)
import jax
import jax.numpy as jnp
from jax.experimental import pallas as pl


def kernel(x, met_sub, rxn_sub, sto_sub, met_all, rxn_all, sto_all, W1, b1, W2, b2, W3, b3, W4, b4, log_k):
    raise NotImplementedError("write your pallas kernel here")


# speedup vs baseline: 14.1292x; 14.1292x over previous
import functools

import jax
import jax.numpy as jnp
from jax import lax
from jax.experimental import pallas as pl
from jax.experimental.pallas import tpu as pltpu
from jax.experimental.pallas import tpu_sc as plsc

N_MET = 50000
N_RXN = 50000
E_SUB = 800000
E_ALL = 1600000
HID = 128
MSG = 32

NC = 2
NS = 16
NW = NC * NS

R_PAD = 50176
E_SUB_PAD = 819200
E_ALL_PAD = 1605632

@functools.cache
def _mesh():
    return plsc.VectorSubcoreMesh(core_axis_name="c", subcore_axis_name="s")


@functools.cache
def _gather_c_kernel():
    return pl.kernel(
        _gather_c_body,
        mesh=_mesh(),
        out_type=jax.ShapeDtypeStruct((E_SUB_PAD // 128, 128), jnp.float32),
        compiler_params=pltpu.CompilerParams(needs_layout_passes=False),
        scratch_types=[
            pltpu.VMEM((R_PAD,), jnp.float32),
            pltpu.VMEM((200, 128), jnp.int32),
            pltpu.VMEM((200, 128), jnp.float32),
        ],
    )


def _gather_c_body(conc_hbm, met_hbm, out_hbm, conc_v, idx_v, out_v):
    c = lax.axis_index("c")
    s = lax.axis_index("s")
    wid = s * NC + c
    base = wid * 200
    pltpu.sync_copy(conc_hbm, conc_v)
    pltpu.sync_copy(met_hbm.at[pl.ds(base, 200)], idx_v)

    def row(r, carry):
        for k in range(8):
            idx = idx_v[r, pl.ds(k * 16, 16)]
            out_v[r, pl.ds(k * 16, 16)] = plsc.load_gather(conc_v, [idx])
        return carry

    lax.fori_loop(0, 200, row, 0)
    pltpu.sync_copy(out_v, out_hbm.at[pl.ds(base, 200)])


def _edge_mlp_body(in_ref, w1t_ref, b1c_ref, w2t_ref, b2c_ref, out_ref):
    h = jnp.tanh(
        jnp.dot(w1t_ref[...], in_ref[...], preferred_element_type=jnp.float32)
        + b1c_ref[...]
    )
    out_ref[...] = (
        jnp.dot(w2t_ref[...], h, preferred_element_type=jnp.float32) + b2c_ref[...]
    )


def _edge_mlp(in_t, W1t, b1c, W2t, b2c):
    blk = 2048
    grid = E_SUB_PAD // blk
    return pl.pallas_call(
        _edge_mlp_body,
        grid=(grid,),
        in_specs=[
            pl.BlockSpec((2, blk), lambda i: (0, i)),
            pl.BlockSpec((HID, 2), lambda i: (0, 0)),
            pl.BlockSpec((HID, 1), lambda i: (0, 0)),
            pl.BlockSpec((MSG, HID), lambda i: (0, 0)),
            pl.BlockSpec((MSG, 1), lambda i: (0, 0)),
        ],
        out_specs=pl.BlockSpec((MSG, blk), lambda i: (0, i)),
        out_shape=jax.ShapeDtypeStruct((MSG, E_SUB_PAD), jnp.float32),
    )(in_t, W1t, b1c, W2t, b2c)


@functools.cache
def _seg_rxn_kernel():
    return pl.kernel(
        _seg_rxn_body,
        mesh=_mesh(),
        out_type=jax.ShapeDtypeStruct((NC * MSG * R_PAD,), jnp.float32),
        compiler_params=pltpu.CompilerParams(needs_layout_passes=False),
        scratch_types=(
            [pltpu.VMEM_SHARED((R_PAD,), jnp.float32)] * MSG
            + [
                pltpu.VMEM((8, 128), jnp.int32),
                pltpu.VMEM((MSG, 128), jnp.float32),
                pltpu.VMEM((3136,), jnp.float32),
            ]
        ),
    )


def _seg_rxn_body(msgt_hbm, rxn_hbm, zero_hbm, out_hbm, *scr):
    planes = scr[:MSG]
    idx_v, msgt_v, zbuf_v = scr[MSG], scr[MSG + 1], scr[MSG + 2]
    c = lax.axis_index("c")
    s = lax.axis_index("s")
    wid = s * NC + c
    pltpu.sync_copy(zero_hbm, zbuf_v)
    for m in range(MSG):
        pltpu.sync_copy(zbuf_v, planes[m].at[pl.ds(s * 3136, 3136)])
    plsc.subcore_barrier()
    ebase = wid * 25600

    def group(g, carry):
        pltpu.sync_copy(rxn_hbm.at[pl.ds(wid * 200 + g * 8, 8)], idx_v)

        def chunk(j, carry2):
            e0 = ebase + g * 1024 + j * 128
            pltpu.sync_copy(msgt_hbm.at[:, pl.ds(e0, 128)], msgt_v)
            for m in range(MSG):
                pltpu.sync_copy(msgt_v.at[m], planes[m].at[idx_v.at[j]],
                                add=True)
            return carry2

        lax.fori_loop(0, 8, chunk, 0)
        return carry

    lax.fori_loop(0, 25, group, 0)
    plsc.subcore_barrier()
    for m in range(MSG):
        pltpu.sync_copy(planes[m].at[pl.ds(s * 3136, 3136)], zbuf_v)
        pltpu.sync_copy(
            zbuf_v,
            out_hbm.at[pl.ds(c * MSG * R_PAD + m * R_PAD + s * 3136, 3136)])


def _rate_body(ha_ref, hb_ref, w3_ref, b3_ref, w4_ref, b4_ref, logk_ref, out_ref):
    h = ha_ref[...] + hb_ref[...]
    t = jnp.tanh(
        jnp.dot(h, w3_ref[...], preferred_element_type=jnp.float32) + b3_ref[...]
    )
    rate = jnp.sum(t * w4_ref[...], axis=1) + b4_ref[0, 0]
    sp = jnp.maximum(rate, 0.0) + jnp.log(1.0 + jnp.exp(-jnp.abs(rate)))
    out_ref[...] = jnp.exp(logk_ref[...] * 2.302585092994046) * sp


def _rate(ha, hb, W3, b3, w4r, b4r, logk):
    blk = 512
    grid = R_PAD // blk
    return pl.pallas_call(
        _rate_body,
        grid=(grid,),
        in_specs=[
            pl.BlockSpec((blk, MSG), lambda i: (i, 0)),
            pl.BlockSpec((blk, MSG), lambda i: (i, 0)),
            pl.BlockSpec((MSG, HID), lambda i: (0, 0)),
            pl.BlockSpec((1, HID), lambda i: (0, 0)),
            pl.BlockSpec((1, HID), lambda i: (0, 0)),
            pl.BlockSpec((1, 1), lambda i: (0, 0)),
            pl.BlockSpec((blk,), lambda i: (i,)),
        ],
        out_specs=pl.BlockSpec((blk,), lambda i: (i,)),
        out_shape=jax.ShapeDtypeStruct((R_PAD,), jnp.float32),
    )(ha, hb, W3, b3, w4r, b4r, logk)


@functools.cache
def _seg_met_kernel():
    return pl.kernel(
        _seg_met_body,
        mesh=_mesh(),
        out_type=jax.ShapeDtypeStruct((NC * R_PAD,), jnp.float32),
        compiler_params=pltpu.CompilerParams(needs_layout_passes=False),
        scratch_types=[
            pltpu.VMEM_SHARED((R_PAD,), jnp.float32),
            pltpu.VMEM((R_PAD,), jnp.float32),
            pltpu.VMEM((3136,), jnp.float32),
            pltpu.VMEM((8, 128), jnp.int32),
            pltpu.VMEM((8, 128), jnp.int32),
            pltpu.VMEM((8, 128), jnp.float32),
            pltpu.VMEM((8, 128), jnp.float32),
        ],
    )


def _seg_met_body(v_hbm, rxn_hbm, met_hbm, sto_hbm, zero_hbm, out_hbm,
                  accum_sh, v_v, zbuf_v, rxn_v, met_v, sto_v, ctr_v):
    c = lax.axis_index("c")
    s = lax.axis_index("s")
    wid = s * NC + c
    pltpu.sync_copy(zero_hbm, zbuf_v)
    pltpu.sync_copy(zbuf_v, accum_sh.at[pl.ds(s * 3136, 3136)])
    pltpu.sync_copy(v_hbm, v_v)
    plsc.subcore_barrier()
    rbase = wid * 392

    def chunk(j, carry):
        r0 = rbase + j * 8
        pltpu.sync_copy(rxn_hbm.at[pl.ds(r0, 8)], rxn_v)
        pltpu.sync_copy(met_hbm.at[pl.ds(r0, 8)], met_v)
        pltpu.sync_copy(sto_hbm.at[pl.ds(r0, 8)], sto_v)
        for k in range(8):
            for i in range(8):
                idx = rxn_v[k, pl.ds(i * 16, 16)]
                vv = plsc.load_gather(v_v, [idx])
                ctr_v[k, pl.ds(i * 16, 16)] = sto_v[k, pl.ds(i * 16, 16)] * vv
        for k in range(8):
            pltpu.sync_copy(ctr_v.at[k], accum_sh.at[met_v.at[k]], add=True)
        return carry

    lax.fori_loop(0, 49, chunk, 0)
    plsc.subcore_barrier()
    pltpu.sync_copy(accum_sh.at[pl.ds(s * 3136, 3136)], zbuf_v)
    pltpu.sync_copy(zbuf_v, out_hbm.at[pl.ds(c * R_PAD + s * 3136, 3136)])


def _addh_body(a_ref, b_ref, o_ref):
    o_ref[...] = a_ref[...] + b_ref[...]


def _addh(a, b):
    return pl.pallas_call(
        _addh_body,
        out_shape=jax.ShapeDtypeStruct((R_PAD,), jnp.float32),
    )(a, b)


def kernel(x, met_sub, rxn_sub, sto_sub, met_all, rxn_all, sto_all,
           W1, b1, W2, b2, W3, b3, W4, b4, log_k):
    i32 = jnp.int32
    f32 = jnp.float32

    conc = jnp.pad(x[:, 3], (0, R_PAD - N_MET))

    ps = E_SUB_PAD - E_SUB
    met_sub_p = jnp.concatenate(
        [met_sub.astype(i32), jnp.arange(ps, dtype=i32) % N_MET])
    rxn_sub_p = jnp.concatenate(
        [rxn_sub.astype(i32), N_RXN + jnp.arange(ps, dtype=i32) % (R_PAD - N_RXN)])
    sto_sub_p = jnp.pad(sto_sub.astype(f32), (0, ps))

    pa = E_ALL_PAD - E_ALL
    met_all_p = jnp.concatenate(
        [met_all.astype(i32), jnp.arange(pa, dtype=i32) % N_MET])
    rxn_all_p = jnp.concatenate(
        [rxn_all.astype(i32), jnp.arange(pa, dtype=i32) % N_RXN])
    sto_all_p = jnp.pad(sto_all.astype(f32), (0, pa))

    c_sub = _gather_c_kernel()(conc, met_sub_p.reshape(-1, 128))

    in_t = jnp.stack([c_sub.reshape(-1), sto_sub_p], axis=0)
    msgt = _edge_mlp(in_t, W1.astype(f32).T, b1.reshape(HID, 1),
                     W2.astype(f32).T, b2.reshape(MSG, 1))

    h2 = _seg_rxn_kernel()(msgt, rxn_sub_p.reshape(-1, 128),
                           jnp.zeros((3136,), dtype=f32))
    h2t = h2.reshape(NC, MSG, R_PAD)

    logk_p = jnp.pad(log_k.astype(f32), (0, R_PAD - N_RXN))
    v = _rate(h2t[0].T, h2t[1].T, W3.astype(f32), b3.reshape(1, HID),
              W4.reshape(1, HID), b4.reshape(1, 1), logk_p)

    dx2 = _seg_met_kernel()(v, rxn_all_p.reshape(-1, 128),
                            met_all_p.reshape(-1, 128),
                            sto_all_p.reshape(-1, 128),
                            jnp.zeros((3136,), dtype=f32))

    dxdt = _addh(dx2[:R_PAD], dx2[R_PAD:])
    return dxdt[:N_MET, None]


# speedup vs baseline: 23.9476x; 1.6949x over previous
import functools

import jax
import jax.numpy as jnp
from jax import lax
from jax.experimental import pallas as pl
from jax.experimental.pallas import tpu as pltpu
from jax.experimental.pallas import tpu_sc as plsc

N_MET = 50000
N_RXN = 50000
E_SUB = 800000
E_ALL = 1600000
HID = 128
MSG = 32

NC = 2
NS = 16
NW = NC * NS

R_PAD = 50176
E_SUB_PAD = 819200
E_ALL_PAD = 1605632

@functools.cache
def _mesh():
    return plsc.VectorSubcoreMesh(core_axis_name="c", subcore_axis_name="s")


@functools.cache
def _gather_c_kernel():
    return pl.kernel(
        _gather_c_body,
        mesh=_mesh(),
        out_type=jax.ShapeDtypeStruct((E_SUB_PAD // 128, 128), jnp.float32),
        compiler_params=pltpu.CompilerParams(needs_layout_passes=False),
        scratch_types=[
            pltpu.VMEM((R_PAD,), jnp.float32),
            pltpu.VMEM((200, 128), jnp.int32),
            pltpu.VMEM((200, 128), jnp.float32),
        ],
    )


def _gather_c_body(conc_hbm, met_hbm, out_hbm, conc_v, idx_v, out_v):
    c = lax.axis_index("c")
    s = lax.axis_index("s")
    wid = s * NC + c
    base = wid * 200
    pltpu.sync_copy(conc_hbm, conc_v)
    pltpu.sync_copy(met_hbm.at[pl.ds(base, 200)], idx_v)

    def row(r, carry):
        for k in range(8):
            idx = idx_v[r, pl.ds(k * 16, 16)]
            out_v[r, pl.ds(k * 16, 16)] = plsc.load_gather(conc_v, [idx])
        return carry

    lax.fori_loop(0, 200, row, 0)
    pltpu.sync_copy(out_v, out_hbm.at[pl.ds(base, 200)])


def _edge_mlp_body(in_ref, w1t_ref, b1c_ref, w2t_ref, b2c_ref, out_ref):
    h = jnp.tanh(
        jnp.dot(w1t_ref[...], in_ref[...], preferred_element_type=jnp.float32)
        + b1c_ref[...]
    )
    out_ref[...] = (
        jnp.dot(w2t_ref[...], h, preferred_element_type=jnp.float32) + b2c_ref[...]
    )


def _edge_mlp(in_t, W1t, b1c, W2t, b2c):
    blk = 2048
    grid = E_SUB_PAD // blk
    return pl.pallas_call(
        _edge_mlp_body,
        grid=(grid,),
        in_specs=[
            pl.BlockSpec((2, blk), lambda i: (0, i)),
            pl.BlockSpec((HID, 2), lambda i: (0, 0)),
            pl.BlockSpec((HID, 1), lambda i: (0, 0)),
            pl.BlockSpec((MSG, HID), lambda i: (0, 0)),
            pl.BlockSpec((MSG, 1), lambda i: (0, 0)),
        ],
        out_specs=pl.BlockSpec((MSG, blk), lambda i: (0, i)),
        out_shape=jax.ShapeDtypeStruct((MSG, E_SUB_PAD), jnp.float32),
    )(in_t, W1t, b1c, W2t, b2c)


@functools.cache
def _seg_rxn_kernel():
    return pl.kernel(
        _seg_rxn_body,
        mesh=_mesh(),
        out_type=jax.ShapeDtypeStruct((NC * MSG * R_PAD,), jnp.float32),
        compiler_params=pltpu.CompilerParams(needs_layout_passes=False),
        scratch_types=(
            [pltpu.VMEM_SHARED((R_PAD,), jnp.float32)] * MSG
            + [
                pltpu.VMEM((8, 128), jnp.int32),
                pltpu.VMEM((MSG, 128), jnp.float32),
                pltpu.VMEM((MSG, 128), jnp.float32),
                pltpu.VMEM((3136,), jnp.float32),
                pltpu.SemaphoreType.DMA,
                pltpu.SemaphoreType.DMA,
            ]
        ),
    )


def _seg_rxn_body(msgt_hbm, rxn_hbm, zero_hbm, out_hbm, *scr):
    planes = scr[:MSG]
    idx_v, buf0, buf1, zbuf_v, lsem, ssem = scr[MSG:]
    bufs = (buf0, buf1)
    c = lax.axis_index("c")
    s = lax.axis_index("s")
    wid = s * NC + c
    pltpu.sync_copy(zero_hbm, zbuf_v)
    zcs = [pltpu.async_copy(zbuf_v, planes[m].at[pl.ds(s * 3136, 3136)], ssem)
           for m in range(MSG)]
    for h in zcs:
        h.wait()
    plsc.subcore_barrier()
    ebase = wid * 25600

    def group(g, carry):
        pltpu.sync_copy(rxn_hbm.at[pl.ds(wid * 200 + g * 8, 8)], idx_v)
        h = pltpu.async_copy(msgt_hbm.at[:, pl.ds(ebase + g * 1024, 128)],
                             bufs[0], lsem)
        for j in range(8):
            h.wait()
            if j < 7:
                e0 = ebase + g * 1024 + (j + 1) * 128
                h = pltpu.async_copy(msgt_hbm.at[:, pl.ds(e0, 128)],
                                     bufs[(j + 1) % 2], lsem)
            scs = [pltpu.async_copy(bufs[j % 2].at[m],
                                    planes[m].at[idx_v.at[j]], ssem, add=True)
                   for m in range(MSG)]
            for sh in scs:
                sh.wait()
        return carry

    lax.fori_loop(0, 25, group, 0)
    plsc.subcore_barrier()
    for m in range(MSG):
        pltpu.sync_copy(planes[m].at[pl.ds(s * 3136, 3136)], zbuf_v)
        pltpu.sync_copy(
            zbuf_v,
            out_hbm.at[pl.ds(c * MSG * R_PAD + m * R_PAD + s * 3136, 3136)])


def _rate_body(ha_ref, hb_ref, w3_ref, b3_ref, w4_ref, b4_ref, logk_ref, out_ref):
    h = ha_ref[...] + hb_ref[...]
    t = jnp.tanh(
        jnp.dot(h, w3_ref[...], preferred_element_type=jnp.float32) + b3_ref[...]
    )
    rate = jnp.sum(t * w4_ref[...], axis=1) + b4_ref[0, 0]
    sp = jnp.maximum(rate, 0.0) + jnp.log(1.0 + jnp.exp(-jnp.abs(rate)))
    out_ref[...] = jnp.exp(logk_ref[...] * 2.302585092994046) * sp


def _rate(ha, hb, W3, b3, w4r, b4r, logk):
    blk = 512
    grid = R_PAD // blk
    return pl.pallas_call(
        _rate_body,
        grid=(grid,),
        in_specs=[
            pl.BlockSpec((blk, MSG), lambda i: (i, 0)),
            pl.BlockSpec((blk, MSG), lambda i: (i, 0)),
            pl.BlockSpec((MSG, HID), lambda i: (0, 0)),
            pl.BlockSpec((1, HID), lambda i: (0, 0)),
            pl.BlockSpec((1, HID), lambda i: (0, 0)),
            pl.BlockSpec((1, 1), lambda i: (0, 0)),
            pl.BlockSpec((blk,), lambda i: (i,)),
        ],
        out_specs=pl.BlockSpec((blk,), lambda i: (i,)),
        out_shape=jax.ShapeDtypeStruct((R_PAD,), jnp.float32),
    )(ha, hb, W3, b3, w4r, b4r, logk)


@functools.cache
def _seg_met_kernel():
    return pl.kernel(
        _seg_met_body,
        mesh=_mesh(),
        out_type=jax.ShapeDtypeStruct((NC * R_PAD,), jnp.float32),
        compiler_params=pltpu.CompilerParams(needs_layout_passes=False),
        scratch_types=[
            pltpu.VMEM_SHARED((R_PAD,), jnp.float32),
            pltpu.VMEM((R_PAD,), jnp.float32),
            pltpu.VMEM((3136,), jnp.float32),
            pltpu.VMEM((8, 128), jnp.int32),
            pltpu.VMEM((8, 128), jnp.int32),
            pltpu.VMEM((8, 128), jnp.float32),
            pltpu.VMEM((8, 128), jnp.float32),
            pltpu.SemaphoreType.DMA,
        ],
    )


def _seg_met_body(v_hbm, rxn_hbm, met_hbm, sto_hbm, zero_hbm, out_hbm,
                  accum_sh, v_v, zbuf_v, rxn_v, met_v, sto_v, ctr_v, lsem):
    c = lax.axis_index("c")
    s = lax.axis_index("s")
    wid = s * NC + c
    pltpu.sync_copy(zero_hbm, zbuf_v)
    pltpu.sync_copy(zbuf_v, accum_sh.at[pl.ds(s * 3136, 3136)])
    pltpu.sync_copy(v_hbm, v_v)
    plsc.subcore_barrier()
    rbase = wid * 392

    def chunk(j, carry):
        r0 = rbase + j * 8
        lcs = [pltpu.async_copy(rxn_hbm.at[pl.ds(r0, 8)], rxn_v, lsem),
               pltpu.async_copy(met_hbm.at[pl.ds(r0, 8)], met_v, lsem),
               pltpu.async_copy(sto_hbm.at[pl.ds(r0, 8)], sto_v, lsem)]
        for h in lcs:
            h.wait()
        for k in range(8):
            for i in range(8):
                idx = rxn_v[k, pl.ds(i * 16, 16)]
                vv = plsc.load_gather(v_v, [idx])
                ctr_v[k, pl.ds(i * 16, 16)] = sto_v[k, pl.ds(i * 16, 16)] * vv
        scs = [pltpu.async_copy(ctr_v.at[k], accum_sh.at[met_v.at[k]], lsem,
                                add=True)
               for k in range(8)]
        for h in scs:
            h.wait()
        return carry

    lax.fori_loop(0, 49, chunk, 0)
    plsc.subcore_barrier()
    pltpu.sync_copy(accum_sh.at[pl.ds(s * 3136, 3136)], zbuf_v)
    pltpu.sync_copy(zbuf_v, out_hbm.at[pl.ds(c * R_PAD + s * 3136, 3136)])


def _addh_body(a_ref, b_ref, o_ref):
    o_ref[...] = a_ref[...] + b_ref[...]


def _addh(a, b):
    return pl.pallas_call(
        _addh_body,
        out_shape=jax.ShapeDtypeStruct((R_PAD,), jnp.float32),
    )(a, b)


def kernel(x, met_sub, rxn_sub, sto_sub, met_all, rxn_all, sto_all,
           W1, b1, W2, b2, W3, b3, W4, b4, log_k):
    i32 = jnp.int32
    f32 = jnp.float32

    conc = jnp.pad(x[:, 3], (0, R_PAD - N_MET))

    ps = E_SUB_PAD - E_SUB
    met_sub_p = jnp.concatenate(
        [met_sub.astype(i32), jnp.arange(ps, dtype=i32) % N_MET])
    rxn_sub_p = jnp.concatenate(
        [rxn_sub.astype(i32), N_RXN + jnp.arange(ps, dtype=i32) % (R_PAD - N_RXN)])
    sto_sub_p = jnp.pad(sto_sub.astype(f32), (0, ps))

    pa = E_ALL_PAD - E_ALL
    met_all_p = jnp.concatenate(
        [met_all.astype(i32), jnp.arange(pa, dtype=i32) % N_MET])
    rxn_all_p = jnp.concatenate(
        [rxn_all.astype(i32), jnp.arange(pa, dtype=i32) % N_RXN])
    sto_all_p = jnp.pad(sto_all.astype(f32), (0, pa))

    c_sub = _gather_c_kernel()(conc, met_sub_p.reshape(-1, 128))

    in_t = jnp.stack([c_sub.reshape(-1), sto_sub_p], axis=0)
    msgt = _edge_mlp(in_t, W1.astype(f32).T, b1.reshape(HID, 1),
                     W2.astype(f32).T, b2.reshape(MSG, 1))

    h2 = _seg_rxn_kernel()(msgt, rxn_sub_p.reshape(-1, 128),
                           jnp.zeros((3136,), dtype=f32))
    h2t = h2.reshape(NC, MSG, R_PAD)

    logk_p = jnp.pad(log_k.astype(f32), (0, R_PAD - N_RXN))
    v = _rate(h2t[0].T, h2t[1].T, W3.astype(f32), b3.reshape(1, HID),
              W4.reshape(1, HID), b4.reshape(1, 1), logk_p)

    dx2 = _seg_met_kernel()(v, rxn_all_p.reshape(-1, 128),
                            met_all_p.reshape(-1, 128),
                            sto_all_p.reshape(-1, 128),
                            jnp.zeros((3136,), dtype=f32))

    dxdt = _addh(dx2[:R_PAD], dx2[R_PAD:])
    return dxdt[:N_MET, None]


# speedup vs baseline: 29.3627x; 1.2261x over previous
import functools

import jax
import jax.numpy as jnp
from jax import lax
from jax.experimental import pallas as pl
from jax.experimental.pallas import tpu as pltpu
from jax.experimental.pallas import tpu_sc as plsc

N_MET = 50000
N_RXN = 50000
E_SUB = 800000
E_ALL = 1600000
HID = 128
MSG = 32

NC = 2
NS = 16
NW = NC * NS

R_PAD = 50176
E_SUB_PAD = 819200
E_ALL_PAD = 1605632

@functools.cache
def _mesh():
    return plsc.VectorSubcoreMesh(core_axis_name="c", subcore_axis_name="s")


@functools.cache
def _gather_c_kernel():
    return pl.kernel(
        _gather_c_body,
        mesh=_mesh(),
        out_type=jax.ShapeDtypeStruct((E_SUB_PAD // 128, 128), jnp.float32),
        compiler_params=pltpu.CompilerParams(needs_layout_passes=False),
        scratch_types=[
            pltpu.VMEM((R_PAD,), jnp.float32),
            pltpu.VMEM((200, 128), jnp.int32),
            pltpu.VMEM((200, 128), jnp.float32),
        ],
    )


def _gather_c_body(conc_hbm, met_hbm, out_hbm, conc_v, idx_v, out_v):
    c = lax.axis_index("c")
    s = lax.axis_index("s")
    wid = s * NC + c
    base = wid * 200
    pltpu.sync_copy(conc_hbm, conc_v)
    pltpu.sync_copy(met_hbm.at[pl.ds(base, 200)], idx_v)

    def row(r, carry):
        for k in range(8):
            idx = idx_v[r, pl.ds(k * 16, 16)]
            out_v[r, pl.ds(k * 16, 16)] = plsc.load_gather(conc_v, [idx])
        return carry

    lax.fori_loop(0, 200, row, 0)
    pltpu.sync_copy(out_v, out_hbm.at[pl.ds(base, 200)])


def _edge_mlp_body(in_ref, w1t_ref, b1c_ref, w2t_ref, b2c_ref, out_ref):
    h = jnp.tanh(
        jnp.dot(w1t_ref[...], in_ref[...], preferred_element_type=jnp.float32)
        + b1c_ref[...]
    )
    o = jnp.dot(w2t_ref[...], h, preferred_element_type=jnp.float32) + b2c_ref[...]
    out_ref[...] = o.reshape(out_ref.shape)


def _edge_mlp(in_t, W1t, b1c, W2t, b2c):
    blkj = 8
    blk = blkj * 2048
    grid = E_SUB_PAD // blk
    return pl.pallas_call(
        _edge_mlp_body,
        grid=(grid,),
        in_specs=[
            pl.BlockSpec((2, blk), lambda i: (0, i)),
            pl.BlockSpec((HID, 2), lambda i: (0, 0)),
            pl.BlockSpec((HID, 1), lambda i: (0, 0)),
            pl.BlockSpec((MSG, HID), lambda i: (0, 0)),
            pl.BlockSpec((MSG, 1), lambda i: (0, 0)),
        ],
        out_specs=pl.BlockSpec((MSG, blkj, 2048), lambda i: (0, i, 0)),
        out_shape=jax.ShapeDtypeStruct((MSG, E_SUB_PAD // 2048, 2048), jnp.float32),
    )(in_t, W1t, b1c, W2t, b2c)


NJ = E_SUB_PAD // 2048
NJ_SC = NJ // NC
CW = 512


@functools.cache
def _seg_rxn_kernel():
    return pl.kernel(
        _seg_rxn_body,
        mesh=_mesh(),
        out_type=jax.ShapeDtypeStruct((NC * MSG * R_PAD,), jnp.float32),
        compiler_params=pltpu.CompilerParams(needs_layout_passes=False),
        scratch_types=[
            pltpu.VMEM((R_PAD,), jnp.float32),
            pltpu.VMEM((R_PAD,), jnp.float32),
            pltpu.VMEM((8, CW), jnp.int32),
            pltpu.VMEM((8, CW), jnp.int32),
            pltpu.VMEM((8, CW), jnp.float32),
            pltpu.VMEM((8, CW), jnp.float32),
            pltpu.VMEM((8, CW), jnp.float32),
            pltpu.VMEM((8, CW), jnp.float32),
            pltpu.SemaphoreType.DMA,
            pltpu.SemaphoreType.DMA,
        ],
    )


def _seg_rxn_body(msgt_hbm, rxn_hbm, zero_hbm, out_hbm,
                  acc0, acc1, idx0, idx1, v0a, v0b, v1a, v1b, sem0, sem1):
    c = lax.axis_index("c")
    s = lax.axis_index("s")
    p0 = 2 * s
    jbase = c * NJ_SC
    idxb = (idx0, idx1)
    vb0 = (v0a, v0b)
    vb1 = (v1a, v1b)
    sems = (sem0, sem1)
    pltpu.sync_copy(zero_hbm, acc0)
    pltpu.sync_copy(zero_hbm, acc1)

    def fire(g, m, b):
        r0 = jbase + g * 8
        m0 = m * CW
        return [
            pltpu.async_copy(rxn_hbm.at[pl.ds(r0, 8), pl.ds(m0, CW)],
                             idxb[b], sems[b]),
            pltpu.async_copy(msgt_hbm.at[p0, pl.ds(r0, 8), pl.ds(m0, CW)],
                             vb0[b], sems[b]),
            pltpu.async_copy(msgt_hbm.at[p0 + 1, pl.ds(r0, 8), pl.ds(m0, CW)],
                             vb1[b], sems[b]),
        ]

    def group(g, carry):
        hs = fire(g, 0, 0)
        for m in range(2048 // CW):
            b = m % 2
            for h in hs:
                h.wait()
            if m < 2048 // CW - 1:
                hs = fire(g, m + 1, 1 - b)

            def row(r, cr):
                for k in range(CW // 16):
                    iv = idxb[b][r, pl.ds(k * 16, 16)]
                    plsc.addupdate_scatter(acc0, [iv],
                                           vb0[b][r, pl.ds(k * 16, 16)])
                    plsc.addupdate_scatter(acc1, [iv],
                                           vb1[b][r, pl.ds(k * 16, 16)])
                return cr

            lax.fori_loop(0, 8, row, 0)
        return carry

    lax.fori_loop(0, NJ_SC // 8, group, 0)
    obase = c * MSG * R_PAD + p0 * R_PAD
    pltpu.sync_copy(acc0, out_hbm.at[pl.ds(obase, R_PAD)])
    pltpu.sync_copy(acc1, out_hbm.at[pl.ds(obase + R_PAD, R_PAD)])


def _rate_body(ha_ref, hb_ref, w3_ref, b3_ref, w4_ref, b4_ref, logk_ref, out_ref):
    h = ha_ref[...] + hb_ref[...]
    t = jnp.tanh(
        jnp.dot(h, w3_ref[...], preferred_element_type=jnp.float32) + b3_ref[...]
    )
    rate = jnp.sum(t * w4_ref[...], axis=1) + b4_ref[0, 0]
    sp = jnp.maximum(rate, 0.0) + jnp.log(1.0 + jnp.exp(-jnp.abs(rate)))
    out_ref[...] = jnp.exp(logk_ref[...] * 2.302585092994046) * sp


def _rate(ha, hb, W3, b3, w4r, b4r, logk):
    blk = 512
    grid = R_PAD // blk
    return pl.pallas_call(
        _rate_body,
        grid=(grid,),
        in_specs=[
            pl.BlockSpec((blk, MSG), lambda i: (i, 0)),
            pl.BlockSpec((blk, MSG), lambda i: (i, 0)),
            pl.BlockSpec((MSG, HID), lambda i: (0, 0)),
            pl.BlockSpec((1, HID), lambda i: (0, 0)),
            pl.BlockSpec((1, HID), lambda i: (0, 0)),
            pl.BlockSpec((1, 1), lambda i: (0, 0)),
            pl.BlockSpec((blk,), lambda i: (i,)),
        ],
        out_specs=pl.BlockSpec((blk,), lambda i: (i,)),
        out_shape=jax.ShapeDtypeStruct((R_PAD,), jnp.float32),
    )(ha, hb, W3, b3, w4r, b4r, logk)


@functools.cache
def _seg_met_kernel():
    return pl.kernel(
        _seg_met_body,
        mesh=_mesh(),
        out_type=jax.ShapeDtypeStruct((NC * R_PAD,), jnp.float32),
        compiler_params=pltpu.CompilerParams(needs_layout_passes=False),
        scratch_types=[
            pltpu.VMEM_SHARED((R_PAD,), jnp.float32),
            pltpu.VMEM((R_PAD,), jnp.float32),
            pltpu.VMEM((3136,), jnp.float32),
            pltpu.VMEM((8, 128), jnp.int32),
            pltpu.VMEM((8, 128), jnp.int32),
            pltpu.VMEM((8, 128), jnp.float32),
            pltpu.VMEM((8, 128), jnp.float32),
            pltpu.SemaphoreType.DMA,
        ],
    )


def _seg_met_body(v_hbm, rxn_hbm, met_hbm, sto_hbm, zero_hbm, out_hbm,
                  accum_sh, v_v, zbuf_v, rxn_v, met_v, sto_v, ctr_v, lsem):
    c = lax.axis_index("c")
    s = lax.axis_index("s")
    wid = s * NC + c
    pltpu.sync_copy(zero_hbm, zbuf_v)
    pltpu.sync_copy(zbuf_v, accum_sh.at[pl.ds(s * 3136, 3136)])
    pltpu.sync_copy(v_hbm, v_v)
    plsc.subcore_barrier()
    rbase = wid * 392

    def chunk(j, carry):
        r0 = rbase + j * 8
        lcs = [pltpu.async_copy(rxn_hbm.at[pl.ds(r0, 8)], rxn_v, lsem),
               pltpu.async_copy(met_hbm.at[pl.ds(r0, 8)], met_v, lsem),
               pltpu.async_copy(sto_hbm.at[pl.ds(r0, 8)], sto_v, lsem)]
        for h in lcs:
            h.wait()
        for k in range(8):
            for i in range(8):
                idx = rxn_v[k, pl.ds(i * 16, 16)]
                vv = plsc.load_gather(v_v, [idx])
                ctr_v[k, pl.ds(i * 16, 16)] = sto_v[k, pl.ds(i * 16, 16)] * vv
        scs = [pltpu.async_copy(ctr_v.at[k], accum_sh.at[met_v.at[k]], lsem,
                                add=True)
               for k in range(8)]
        for h in scs:
            h.wait()
        return carry

    lax.fori_loop(0, 49, chunk, 0)
    plsc.subcore_barrier()
    pltpu.sync_copy(accum_sh.at[pl.ds(s * 3136, 3136)], zbuf_v)
    pltpu.sync_copy(zbuf_v, out_hbm.at[pl.ds(c * R_PAD + s * 3136, 3136)])


def _addh_body(a_ref, b_ref, o_ref):
    o_ref[...] = a_ref[...] + b_ref[...]


def _addh(a, b):
    return pl.pallas_call(
        _addh_body,
        out_shape=jax.ShapeDtypeStruct((R_PAD,), jnp.float32),
    )(a, b)


def kernel(x, met_sub, rxn_sub, sto_sub, met_all, rxn_all, sto_all,
           W1, b1, W2, b2, W3, b3, W4, b4, log_k):
    i32 = jnp.int32
    f32 = jnp.float32

    conc = jnp.pad(x[:, 3], (0, R_PAD - N_MET))

    ps = E_SUB_PAD - E_SUB
    met_sub_p = jnp.concatenate(
        [met_sub.astype(i32), jnp.arange(ps, dtype=i32) % N_MET])
    rxn_sub_p = jnp.concatenate(
        [rxn_sub.astype(i32), N_RXN + jnp.arange(ps, dtype=i32) % (R_PAD - N_RXN)])
    sto_sub_p = jnp.pad(sto_sub.astype(f32), (0, ps))

    pa = E_ALL_PAD - E_ALL
    met_all_p = jnp.concatenate(
        [met_all.astype(i32), jnp.arange(pa, dtype=i32) % N_MET])
    rxn_all_p = jnp.concatenate(
        [rxn_all.astype(i32), jnp.arange(pa, dtype=i32) % N_RXN])
    sto_all_p = jnp.pad(sto_all.astype(f32), (0, pa))

    c_sub = _gather_c_kernel()(conc, met_sub_p.reshape(-1, 128))

    in_t = jnp.stack([c_sub.reshape(-1), sto_sub_p], axis=0)
    msgt = _edge_mlp(in_t, W1.astype(f32).T, b1.reshape(HID, 1),
                     W2.astype(f32).T, b2.reshape(MSG, 1))

    h2 = _seg_rxn_kernel()(msgt, rxn_sub_p.reshape(NJ, 2048),
                           jnp.zeros((R_PAD,), dtype=f32))
    h2t = h2.reshape(NC, MSG, R_PAD)

    logk_p = jnp.pad(log_k.astype(f32), (0, R_PAD - N_RXN))
    v = _rate(h2t[0].T, h2t[1].T, W3.astype(f32), b3.reshape(1, HID),
              W4.reshape(1, HID), b4.reshape(1, 1), logk_p)

    dx2 = _seg_met_kernel()(v, rxn_all_p.reshape(-1, 128),
                            met_all_p.reshape(-1, 128),
                            sto_all_p.reshape(-1, 128),
                            jnp.zeros((3136,), dtype=f32))

    dxdt = _addh(dx2[:R_PAD], dx2[R_PAD:])
    return dxdt[:N_MET, None]


# speedup vs baseline: 37.2670x; 1.2692x over previous
import functools

import jax
import jax.numpy as jnp
from jax import lax
from jax.experimental import pallas as pl
from jax.experimental.pallas import tpu as pltpu
from jax.experimental.pallas import tpu_sc as plsc

N_MET = 50000
N_RXN = 50000
E_SUB = 800000
E_ALL = 1600000
HID = 128
MSG = 32

NC = 2
NS = 16
NW = NC * NS

R_PAD = 50176
E_SUB_PAD = 819200
E_ALL_PAD = 1605632

@functools.cache
def _mesh():
    return plsc.VectorSubcoreMesh(core_axis_name="c", subcore_axis_name="s")


@functools.cache
def _gather_c_kernel():
    return pl.kernel(
        _gather_c_body,
        mesh=_mesh(),
        out_type=jax.ShapeDtypeStruct((E_SUB_PAD // 128, 128), jnp.float32),
        compiler_params=pltpu.CompilerParams(needs_layout_passes=False),
        scratch_types=[
            pltpu.VMEM((R_PAD,), jnp.float32),
            pltpu.VMEM((200, 128), jnp.int32),
            pltpu.VMEM((200, 128), jnp.float32),
        ],
    )


def _gather_c_body(conc_hbm, met_hbm, out_hbm, conc_v, idx_v, out_v):
    c = lax.axis_index("c")
    s = lax.axis_index("s")
    wid = s * NC + c
    base = wid * 200
    pltpu.sync_copy(conc_hbm, conc_v)
    pltpu.sync_copy(met_hbm.at[pl.ds(base, 200)], idx_v)

    def row(r, carry):
        for k in range(8):
            idx = idx_v[r, pl.ds(k * 16, 16)]
            out_v[r, pl.ds(k * 16, 16)] = plsc.load_gather(conc_v, [idx])
        return carry

    lax.fori_loop(0, 200, row, 0)
    pltpu.sync_copy(out_v, out_hbm.at[pl.ds(base, 200)])


def _edge_mlp_body(in_ref, w1t_ref, b1c_ref, w2t_ref, b2c_ref, out_ref):
    h = jnp.tanh(
        jnp.dot(w1t_ref[...], in_ref[...], preferred_element_type=jnp.float32)
        + b1c_ref[...]
    )
    o = jnp.dot(w2t_ref[...], h, preferred_element_type=jnp.float32) + b2c_ref[...]
    out_ref[...] = o.reshape(out_ref.shape)


def _edge_mlp(in_t, W1t, b1c, W2t, b2c):
    blkj = 8
    blk = blkj * 2048
    grid = E_SUB_PAD // blk
    return pl.pallas_call(
        _edge_mlp_body,
        grid=(grid,),
        in_specs=[
            pl.BlockSpec((2, blk), lambda i: (0, i)),
            pl.BlockSpec((HID, 2), lambda i: (0, 0)),
            pl.BlockSpec((HID, 1), lambda i: (0, 0)),
            pl.BlockSpec((MSG, HID), lambda i: (0, 0)),
            pl.BlockSpec((MSG, 1), lambda i: (0, 0)),
        ],
        out_specs=pl.BlockSpec((MSG, blkj, 2048), lambda i: (0, i, 0)),
        out_shape=jax.ShapeDtypeStruct((MSG, E_SUB_PAD // 2048, 2048), jnp.float32),
    )(in_t, W1t, b1c, W2t, b2c)


NJ = E_SUB_PAD // 2048
NJ_SC = NJ // NC
CW = 512


@functools.cache
def _seg_rxn_kernel():
    return pl.kernel(
        _seg_rxn_body,
        mesh=_mesh(),
        out_type=jax.ShapeDtypeStruct((NC * MSG * R_PAD,), jnp.float32),
        compiler_params=pltpu.CompilerParams(needs_layout_passes=False),
        scratch_types=[
            pltpu.VMEM((R_PAD,), jnp.float32),
            pltpu.VMEM((R_PAD,), jnp.float32),
            pltpu.VMEM((8, CW), jnp.int32),
            pltpu.VMEM((8, CW), jnp.int32),
            pltpu.VMEM((8, CW), jnp.float32),
            pltpu.VMEM((8, CW), jnp.float32),
            pltpu.VMEM((8, CW), jnp.float32),
            pltpu.VMEM((8, CW), jnp.float32),
            pltpu.SemaphoreType.DMA,
            pltpu.SemaphoreType.DMA,
        ],
    )


def _seg_rxn_body(msgt_hbm, rxn_hbm, zero_hbm, out_hbm,
                  acc0, acc1, idx0, idx1, v0a, v0b, v1a, v1b, sem0, sem1):
    c = lax.axis_index("c")
    s = lax.axis_index("s")
    p0 = 2 * s
    jbase = c * NJ_SC
    idxb = (idx0, idx1)
    vb0 = (v0a, v0b)
    vb1 = (v1a, v1b)
    sems = (sem0, sem1)
    pltpu.sync_copy(zero_hbm, acc0)
    pltpu.sync_copy(zero_hbm, acc1)

    def fire(g, m, b):
        r0 = jbase + g * 8
        m0 = m * CW
        return [
            pltpu.async_copy(rxn_hbm.at[pl.ds(r0, 8), pl.ds(m0, CW)],
                             idxb[b], sems[b]),
            pltpu.async_copy(msgt_hbm.at[p0, pl.ds(r0, 8), pl.ds(m0, CW)],
                             vb0[b], sems[b]),
            pltpu.async_copy(msgt_hbm.at[p0 + 1, pl.ds(r0, 8), pl.ds(m0, CW)],
                             vb1[b], sems[b]),
        ]

    def group(g, carry):
        hs = fire(g, 0, 0)
        for m in range(2048 // CW):
            b = m % 2
            for h in hs:
                h.wait()
            if m < 2048 // CW - 1:
                hs = fire(g, m + 1, 1 - b)

            nk = CW // 16

            def vec(t, cr):
                r = t // nk
                k16 = (t % nk) * 16
                iv = idxb[b][r, pl.ds(k16, 16)]
                plsc.addupdate_scatter(acc0, [iv],
                                       vb0[b][r, pl.ds(k16, 16)])
                plsc.addupdate_scatter(acc1, [iv],
                                       vb1[b][r, pl.ds(k16, 16)])
                return cr

            plsc.parallel_loop(0, 8 * nk, unroll=4, carry=jnp.int32(0))(vec)
        return carry

    lax.fori_loop(0, NJ_SC // 8, group, 0)
    obase = c * MSG * R_PAD + p0 * R_PAD
    pltpu.sync_copy(acc0, out_hbm.at[pl.ds(obase, R_PAD)])
    pltpu.sync_copy(acc1, out_hbm.at[pl.ds(obase + R_PAD, R_PAD)])


def _rate_body(ha_ref, hb_ref, w3_ref, b3_ref, w4_ref, b4_ref, logk_ref, out_ref):
    h = ha_ref[...] + hb_ref[...]
    t = jnp.tanh(
        jnp.dot(h, w3_ref[...], preferred_element_type=jnp.float32) + b3_ref[...]
    )
    rate = jnp.sum(t * w4_ref[...], axis=1) + b4_ref[0, 0]
    sp = jnp.maximum(rate, 0.0) + jnp.log(1.0 + jnp.exp(-jnp.abs(rate)))
    out_ref[...] = jnp.exp(logk_ref[...] * 2.302585092994046) * sp


def _rate(ha, hb, W3, b3, w4r, b4r, logk):
    blk = 512
    grid = R_PAD // blk
    return pl.pallas_call(
        _rate_body,
        grid=(grid,),
        in_specs=[
            pl.BlockSpec((blk, MSG), lambda i: (i, 0)),
            pl.BlockSpec((blk, MSG), lambda i: (i, 0)),
            pl.BlockSpec((MSG, HID), lambda i: (0, 0)),
            pl.BlockSpec((1, HID), lambda i: (0, 0)),
            pl.BlockSpec((1, HID), lambda i: (0, 0)),
            pl.BlockSpec((1, 1), lambda i: (0, 0)),
            pl.BlockSpec((blk,), lambda i: (i,)),
        ],
        out_specs=pl.BlockSpec((blk,), lambda i: (i,)),
        out_shape=jax.ShapeDtypeStruct((R_PAD,), jnp.float32),
    )(ha, hb, W3, b3, w4r, b4r, logk)


@functools.cache
def _seg_met_kernel():
    return pl.kernel(
        _seg_met_body,
        mesh=_mesh(),
        out_type=jax.ShapeDtypeStruct((NC * R_PAD,), jnp.float32),
        compiler_params=pltpu.CompilerParams(needs_layout_passes=False),
        scratch_types=[
            pltpu.VMEM_SHARED((R_PAD,), jnp.float32),
            pltpu.VMEM((R_PAD,), jnp.float32),
            pltpu.VMEM((3136,), jnp.float32),
            pltpu.VMEM((8, 128), jnp.int32),
            pltpu.VMEM((8, 128), jnp.int32),
            pltpu.VMEM((8, 128), jnp.float32),
            pltpu.VMEM((8, 128), jnp.float32),
            pltpu.SemaphoreType.DMA,
        ],
    )


def _seg_met_body(v_hbm, rxn_hbm, met_hbm, sto_hbm, zero_hbm, out_hbm,
                  accum_sh, v_v, zbuf_v, rxn_v, met_v, sto_v, ctr_v, lsem):
    c = lax.axis_index("c")
    s = lax.axis_index("s")
    wid = s * NC + c
    pltpu.sync_copy(zero_hbm, zbuf_v)
    pltpu.sync_copy(zbuf_v, accum_sh.at[pl.ds(s * 3136, 3136)])
    pltpu.sync_copy(v_hbm, v_v)
    plsc.subcore_barrier()
    rbase = wid * 392

    def chunk(j, carry):
        r0 = rbase + j * 8
        lcs = [pltpu.async_copy(rxn_hbm.at[pl.ds(r0, 8)], rxn_v, lsem),
               pltpu.async_copy(met_hbm.at[pl.ds(r0, 8)], met_v, lsem),
               pltpu.async_copy(sto_hbm.at[pl.ds(r0, 8)], sto_v, lsem)]
        for h in lcs:
            h.wait()
        for k in range(8):
            for i in range(8):
                idx = rxn_v[k, pl.ds(i * 16, 16)]
                vv = plsc.load_gather(v_v, [idx])
                ctr_v[k, pl.ds(i * 16, 16)] = sto_v[k, pl.ds(i * 16, 16)] * vv
        scs = [pltpu.async_copy(ctr_v.at[k], accum_sh.at[met_v.at[k]], lsem,
                                add=True)
               for k in range(8)]
        for h in scs:
            h.wait()
        return carry

    lax.fori_loop(0, 49, chunk, 0)
    plsc.subcore_barrier()
    pltpu.sync_copy(accum_sh.at[pl.ds(s * 3136, 3136)], zbuf_v)
    pltpu.sync_copy(zbuf_v, out_hbm.at[pl.ds(c * R_PAD + s * 3136, 3136)])


def _addh_body(a_ref, b_ref, o_ref):
    o_ref[...] = a_ref[...] + b_ref[...]


def _addh(a, b):
    return pl.pallas_call(
        _addh_body,
        out_shape=jax.ShapeDtypeStruct((R_PAD,), jnp.float32),
    )(a, b)


def kernel(x, met_sub, rxn_sub, sto_sub, met_all, rxn_all, sto_all,
           W1, b1, W2, b2, W3, b3, W4, b4, log_k):
    i32 = jnp.int32
    f32 = jnp.float32

    conc = jnp.pad(x[:, 3], (0, R_PAD - N_MET))

    ps = E_SUB_PAD - E_SUB
    met_sub_p = jnp.concatenate(
        [met_sub.astype(i32), jnp.arange(ps, dtype=i32) % N_MET])
    rxn_sub_p = jnp.concatenate(
        [rxn_sub.astype(i32), N_RXN + jnp.arange(ps, dtype=i32) % (R_PAD - N_RXN)])
    sto_sub_p = jnp.pad(sto_sub.astype(f32), (0, ps))

    pa = E_ALL_PAD - E_ALL
    met_all_p = jnp.concatenate(
        [met_all.astype(i32), jnp.arange(pa, dtype=i32) % N_MET])
    rxn_all_p = jnp.concatenate(
        [rxn_all.astype(i32), jnp.arange(pa, dtype=i32) % N_RXN])
    sto_all_p = jnp.pad(sto_all.astype(f32), (0, pa))

    c_sub = _gather_c_kernel()(conc, met_sub_p.reshape(-1, 128))

    in_t = jnp.stack([c_sub.reshape(-1), sto_sub_p], axis=0)
    msgt = _edge_mlp(in_t, W1.astype(f32).T, b1.reshape(HID, 1),
                     W2.astype(f32).T, b2.reshape(MSG, 1))

    h2 = _seg_rxn_kernel()(msgt, rxn_sub_p.reshape(NJ, 2048),
                           jnp.zeros((R_PAD,), dtype=f32))
    h2t = h2.reshape(NC, MSG, R_PAD)

    logk_p = jnp.pad(log_k.astype(f32), (0, R_PAD - N_RXN))
    v = _rate(h2t[0].T, h2t[1].T, W3.astype(f32), b3.reshape(1, HID),
              W4.reshape(1, HID), b4.reshape(1, 1), logk_p)

    dx2 = _seg_met_kernel()(v, rxn_all_p.reshape(-1, 128),
                            met_all_p.reshape(-1, 128),
                            sto_all_p.reshape(-1, 128),
                            jnp.zeros((3136,), dtype=f32))

    dxdt = _addh(dx2[:R_PAD], dx2[R_PAD:])
    return dxdt[:N_MET, None]


# speedup vs baseline: 38.0572x; 1.0212x over previous
import functools

import jax
import jax.numpy as jnp
from jax import lax
from jax.experimental import pallas as pl
from jax.experimental.pallas import tpu as pltpu
from jax.experimental.pallas import tpu_sc as plsc

N_MET = 50000
N_RXN = 50000
E_SUB = 800000
E_ALL = 1600000
HID = 128
MSG = 32

NC = 2
NS = 16
NW = NC * NS

R_PAD = 50176
E_SUB_PAD = 819200
E_ALL_PAD = 1605632

@functools.cache
def _mesh():
    return plsc.VectorSubcoreMesh(core_axis_name="c", subcore_axis_name="s")


@functools.cache
def _gather_c_kernel():
    return pl.kernel(
        _gather_c_body,
        mesh=_mesh(),
        out_type=jax.ShapeDtypeStruct((E_SUB_PAD // 128, 128), jnp.float32),
        compiler_params=pltpu.CompilerParams(needs_layout_passes=False),
        scratch_types=[
            pltpu.VMEM((R_PAD,), jnp.float32),
            pltpu.VMEM((200, 128), jnp.int32),
            pltpu.VMEM((200, 128), jnp.float32),
        ],
    )


def _gather_c_body(conc_hbm, met_hbm, out_hbm, conc_v, idx_v, out_v):
    c = lax.axis_index("c")
    s = lax.axis_index("s")
    wid = s * NC + c
    base = wid * 200
    pltpu.sync_copy(conc_hbm, conc_v)
    pltpu.sync_copy(met_hbm.at[pl.ds(base, 200)], idx_v)

    def row(r, carry):
        for k in range(8):
            idx = idx_v[r, pl.ds(k * 16, 16)]
            out_v[r, pl.ds(k * 16, 16)] = plsc.load_gather(conc_v, [idx])
        return carry

    lax.fori_loop(0, 200, row, 0)
    pltpu.sync_copy(out_v, out_hbm.at[pl.ds(base, 200)])


def _edge_mlp_body(in_ref, w1t_ref, b1c_ref, w2t_ref, b2c_ref, out_ref):
    h = jnp.tanh(
        jnp.dot(w1t_ref[...], in_ref[...], preferred_element_type=jnp.float32)
        + b1c_ref[...]
    )
    o = jnp.dot(w2t_ref[...], h, preferred_element_type=jnp.float32) + b2c_ref[...]
    out_ref[...] = o.reshape(out_ref.shape)


def _edge_mlp(in_t, W1t, b1c, W2t, b2c):
    blkj = 8
    blk = blkj * 2048
    grid = E_SUB_PAD // blk
    return pl.pallas_call(
        _edge_mlp_body,
        grid=(grid,),
        in_specs=[
            pl.BlockSpec((2, blk), lambda i: (0, i)),
            pl.BlockSpec((HID, 2), lambda i: (0, 0)),
            pl.BlockSpec((HID, 1), lambda i: (0, 0)),
            pl.BlockSpec((MSG, HID), lambda i: (0, 0)),
            pl.BlockSpec((MSG, 1), lambda i: (0, 0)),
        ],
        out_specs=pl.BlockSpec((MSG, blkj, 2048), lambda i: (0, i, 0)),
        out_shape=jax.ShapeDtypeStruct((MSG, E_SUB_PAD // 2048, 2048), jnp.float32),
    )(in_t, W1t, b1c, W2t, b2c)


NJ = E_SUB_PAD // 2048
NJ_SC = NJ // NC
CW = 512


@functools.cache
def _seg_rxn_kernel():
    return pl.kernel(
        _seg_rxn_body,
        mesh=_mesh(),
        out_type=jax.ShapeDtypeStruct((NC * MSG * R_PAD,), jnp.float32),
        compiler_params=pltpu.CompilerParams(needs_layout_passes=False),
        scratch_types=[
            pltpu.VMEM((R_PAD,), jnp.float32),
            pltpu.VMEM((R_PAD,), jnp.float32),
            pltpu.VMEM((8, CW), jnp.int32),
            pltpu.VMEM((8, CW), jnp.int32),
            pltpu.VMEM((8, CW), jnp.float32),
            pltpu.VMEM((8, CW), jnp.float32),
            pltpu.VMEM((8, CW), jnp.float32),
            pltpu.VMEM((8, CW), jnp.float32),
            pltpu.SemaphoreType.DMA,
            pltpu.SemaphoreType.DMA,
        ],
    )


def _seg_rxn_body(msgt_hbm, rxn_hbm, zero_hbm, out_hbm,
                  acc0, acc1, idx0, idx1, v0a, v0b, v1a, v1b, sem0, sem1):
    c = lax.axis_index("c")
    s = lax.axis_index("s")
    p0 = 2 * s
    jbase = c * NJ_SC
    idxb = (idx0, idx1)
    vb0 = (v0a, v0b)
    vb1 = (v1a, v1b)
    sems = (sem0, sem1)
    pltpu.sync_copy(zero_hbm, acc0)
    pltpu.sync_copy(zero_hbm, acc1)

    def fire(g, m, b):
        r0 = jbase + g * 8
        m0 = m * CW
        return [
            pltpu.async_copy(rxn_hbm.at[pl.ds(r0, 8), pl.ds(m0, CW)],
                             idxb[b], sems[b]),
            pltpu.async_copy(msgt_hbm.at[p0, pl.ds(r0, 8), pl.ds(m0, CW)],
                             vb0[b], sems[b]),
            pltpu.async_copy(msgt_hbm.at[p0 + 1, pl.ds(r0, 8), pl.ds(m0, CW)],
                             vb1[b], sems[b]),
        ]

    def group(g, carry):
        hs = fire(g, 0, 0)
        for m in range(2048 // CW):
            b = m % 2
            for h in hs:
                h.wait()
            if m < 2048 // CW - 1:
                hs = fire(g, m + 1, 1 - b)

            nk = CW // 16

            def vec(t, cr):
                r = t // nk
                k16 = (t % nk) * 16
                iv = idxb[b][r, pl.ds(k16, 16)]
                plsc.addupdate_scatter(acc0, [iv],
                                       vb0[b][r, pl.ds(k16, 16)])
                plsc.addupdate_scatter(acc1, [iv],
                                       vb1[b][r, pl.ds(k16, 16)])
                return cr

            plsc.parallel_loop(0, 8 * nk, unroll=4, carry=jnp.int32(0))(vec)
        return carry

    lax.fori_loop(0, NJ_SC // 8, group, 0)
    obase = c * MSG * R_PAD + p0 * R_PAD
    pltpu.sync_copy(acc0, out_hbm.at[pl.ds(obase, R_PAD)])
    pltpu.sync_copy(acc1, out_hbm.at[pl.ds(obase + R_PAD, R_PAD)])


def _rate_body(ha_ref, hb_ref, w3_ref, b3_ref, w4_ref, b4_ref, logk_ref, out_ref):
    h = ha_ref[...] + hb_ref[...]
    t = jnp.tanh(
        jnp.dot(h, w3_ref[...], preferred_element_type=jnp.float32) + b3_ref[...]
    )
    rate = jnp.sum(t * w4_ref[...], axis=1) + b4_ref[0, 0]
    sp = jnp.maximum(rate, 0.0) + jnp.log(1.0 + jnp.exp(-jnp.abs(rate)))
    out_ref[...] = jnp.exp(logk_ref[...] * 2.302585092994046) * sp


def _rate(ha, hb, W3, b3, w4r, b4r, logk):
    blk = 512
    grid = R_PAD // blk
    return pl.pallas_call(
        _rate_body,
        grid=(grid,),
        in_specs=[
            pl.BlockSpec((blk, MSG), lambda i: (i, 0)),
            pl.BlockSpec((blk, MSG), lambda i: (i, 0)),
            pl.BlockSpec((MSG, HID), lambda i: (0, 0)),
            pl.BlockSpec((1, HID), lambda i: (0, 0)),
            pl.BlockSpec((1, HID), lambda i: (0, 0)),
            pl.BlockSpec((1, 1), lambda i: (0, 0)),
            pl.BlockSpec((blk,), lambda i: (i,)),
        ],
        out_specs=pl.BlockSpec((blk,), lambda i: (i,)),
        out_shape=jax.ShapeDtypeStruct((R_PAD,), jnp.float32),
    )(ha, hb, W3, b3, w4r, b4r, logk)


@functools.cache
def _seg_met_kernel():
    return pl.kernel(
        _seg_met_body,
        mesh=_mesh(),
        out_type=jax.ShapeDtypeStruct((NW * R_PAD,), jnp.float32),
        compiler_params=pltpu.CompilerParams(needs_layout_passes=False),
        scratch_types=[
            pltpu.VMEM((R_PAD,), jnp.float32),
            pltpu.VMEM((R_PAD,), jnp.float32),
            pltpu.VMEM((8, 128), jnp.int32),
            pltpu.VMEM((8, 128), jnp.int32),
            pltpu.VMEM((8, 128), jnp.int32),
            pltpu.VMEM((8, 128), jnp.int32),
            pltpu.VMEM((8, 128), jnp.float32),
            pltpu.VMEM((8, 128), jnp.float32),
            pltpu.SemaphoreType.DMA,
            pltpu.SemaphoreType.DMA,
        ],
    )


def _seg_met_body(v_hbm, rxn_hbm, met_hbm, sto_hbm, zero_hbm, out_hbm,
                  acc, v_v, rxn0, rxn1, met0, met1, sto0, sto1, sem0, sem1):
    c = lax.axis_index("c")
    s = lax.axis_index("s")
    wid = s * NC + c
    rxnb = (rxn0, rxn1)
    metb = (met0, met1)
    stob = (sto0, sto1)
    sems = (sem0, sem1)
    pltpu.sync_copy(zero_hbm, acc)
    pltpu.sync_copy(v_hbm, v_v)
    rbase = wid * 392

    def fire(g, b):
        r0 = rbase + g * 8
        return [pltpu.async_copy(rxn_hbm.at[pl.ds(r0, 8)], rxnb[b], sems[b]),
                pltpu.async_copy(met_hbm.at[pl.ds(r0, 8)], metb[b], sems[b]),
                pltpu.async_copy(sto_hbm.at[pl.ds(r0, 8)], stob[b], sems[b])]

    hs = fire(0, 0)
    for g in range(49):
        b = g % 2
        for h in hs:
            h.wait()
        if g < 48:
            hs = fire(g + 1, 1 - b)

        def vec(t, cr, b=b):
            r = t // 8
            k16 = (t % 8) * 16
            rx = rxnb[b][r, pl.ds(k16, 16)]
            mt = metb[b][r, pl.ds(k16, 16)]
            st = stob[b][r, pl.ds(k16, 16)]
            vv = plsc.load_gather(v_v, [rx])
            plsc.addupdate_scatter(acc, [mt], st * vv)
            return cr

        plsc.parallel_loop(0, 64, unroll=4, carry=jnp.int32(0))(vec)

    pltpu.sync_copy(acc, out_hbm.at[pl.ds(wid * R_PAD, R_PAD)])


def _sumw_body(a_ref, o_ref):
    o_ref[...] = jnp.sum(a_ref[...], axis=0)


def _sumw(a):
    blk = 7168
    return pl.pallas_call(
        _sumw_body,
        grid=(R_PAD // blk,),
        in_specs=[pl.BlockSpec((NW, blk), lambda i: (0, i))],
        out_specs=pl.BlockSpec((blk,), lambda i: (i,)),
        out_shape=jax.ShapeDtypeStruct((R_PAD,), jnp.float32),
    )(a)


def kernel(x, met_sub, rxn_sub, sto_sub, met_all, rxn_all, sto_all,
           W1, b1, W2, b2, W3, b3, W4, b4, log_k):
    i32 = jnp.int32
    f32 = jnp.float32

    conc = jnp.pad(x[:, 3], (0, R_PAD - N_MET))

    ps = E_SUB_PAD - E_SUB
    met_sub_p = jnp.concatenate(
        [met_sub.astype(i32), jnp.arange(ps, dtype=i32) % N_MET])
    rxn_sub_p = jnp.concatenate(
        [rxn_sub.astype(i32), N_RXN + jnp.arange(ps, dtype=i32) % (R_PAD - N_RXN)])
    sto_sub_p = jnp.pad(sto_sub.astype(f32), (0, ps))

    pa = E_ALL_PAD - E_ALL
    met_all_p = jnp.concatenate(
        [met_all.astype(i32), jnp.arange(pa, dtype=i32) % N_MET])
    rxn_all_p = jnp.concatenate(
        [rxn_all.astype(i32), jnp.arange(pa, dtype=i32) % N_RXN])
    sto_all_p = jnp.pad(sto_all.astype(f32), (0, pa))

    c_sub = _gather_c_kernel()(conc, met_sub_p.reshape(-1, 128))

    in_t = jnp.stack([c_sub.reshape(-1), sto_sub_p], axis=0)
    msgt = _edge_mlp(in_t, W1.astype(f32).T, b1.reshape(HID, 1),
                     W2.astype(f32).T, b2.reshape(MSG, 1))

    h2 = _seg_rxn_kernel()(msgt, rxn_sub_p.reshape(NJ, 2048),
                           jnp.zeros((R_PAD,), dtype=f32))
    h2t = h2.reshape(NC, MSG, R_PAD)

    logk_p = jnp.pad(log_k.astype(f32), (0, R_PAD - N_RXN))
    v = _rate(h2t[0].T, h2t[1].T, W3.astype(f32), b3.reshape(1, HID),
              W4.reshape(1, HID), b4.reshape(1, 1), logk_p)

    dx32 = _seg_met_kernel()(v, rxn_all_p.reshape(-1, 128),
                             met_all_p.reshape(-1, 128),
                             sto_all_p.reshape(-1, 128),
                             jnp.zeros((R_PAD,), dtype=f32))

    dxdt = _sumw(dx32.reshape(NW, R_PAD))
    return dxdt[:N_MET, None]


# speedup vs baseline: 44.6964x; 1.1745x over previous
import functools

import jax
import jax.numpy as jnp
from jax import lax
from jax.experimental import pallas as pl
from jax.experimental.pallas import tpu as pltpu
from jax.experimental.pallas import tpu_sc as plsc

N_MET = 50000
N_RXN = 50000
E_SUB = 800000
E_ALL = 1600000
HID = 128
MSG = 32

NC = 2
NS = 16
NW = NC * NS

R_PAD = 50176
E_SUB_PAD = 819200
E_ALL_PAD = 1605632

@functools.cache
def _mesh():
    return plsc.VectorSubcoreMesh(core_axis_name="c", subcore_axis_name="s")


@functools.cache
def _gather_c_kernel():
    return pl.kernel(
        _gather_c_body,
        mesh=_mesh(),
        out_type=jax.ShapeDtypeStruct((E_SUB_PAD // 128, 128), jnp.float32),
        compiler_params=pltpu.CompilerParams(needs_layout_passes=False),
        scratch_types=[
            pltpu.VMEM((R_PAD,), jnp.float32),
            pltpu.VMEM((200, 128), jnp.int32),
            pltpu.VMEM((200, 128), jnp.float32),
        ],
    )


def _gather_c_body(conc_hbm, met_hbm, out_hbm, conc_v, idx_v, out_v):
    c = lax.axis_index("c")
    s = lax.axis_index("s")
    wid = s * NC + c
    base = wid * 200
    pltpu.sync_copy(conc_hbm, conc_v)
    pltpu.sync_copy(met_hbm.at[pl.ds(base, 200)], idx_v)

    def row(r, carry):
        for k in range(8):
            idx = idx_v[r, pl.ds(k * 16, 16)]
            out_v[r, pl.ds(k * 16, 16)] = plsc.load_gather(conc_v, [idx])
        return carry

    lax.fori_loop(0, 200, row, 0)
    pltpu.sync_copy(out_v, out_hbm.at[pl.ds(base, 200)])


def _edge_mlp_body(in_ref, w1t_ref, b1c_ref, w2t_ref, b2c_ref, out_ref):
    h = jnp.tanh(
        jnp.dot(w1t_ref[...], in_ref[...], preferred_element_type=jnp.float32)
        + b1c_ref[...]
    )
    o = jnp.dot(w2t_ref[...], h, preferred_element_type=jnp.float32) + b2c_ref[...]
    out_ref[...] = o.reshape(out_ref.shape)


def _edge_mlp(in_t, W1t, b1c, W2t, b2c):
    blkj = 8
    blk = blkj * 2048
    grid = E_SUB_PAD // blk
    return pl.pallas_call(
        _edge_mlp_body,
        grid=(grid,),
        in_specs=[
            pl.BlockSpec((2, blk), lambda i: (0, i)),
            pl.BlockSpec((HID, 2), lambda i: (0, 0)),
            pl.BlockSpec((HID, 1), lambda i: (0, 0)),
            pl.BlockSpec((MSG, HID), lambda i: (0, 0)),
            pl.BlockSpec((MSG, 1), lambda i: (0, 0)),
        ],
        out_specs=pl.BlockSpec((MSG, blkj, 2048), lambda i: (0, i, 0)),
        out_shape=jax.ShapeDtypeStruct((MSG, E_SUB_PAD // 2048, 2048), jnp.float32),
    )(in_t, W1t, b1c, W2t, b2c)


NJ = E_SUB_PAD // 2048
NJ_SC = NJ // NC
CW = 512


@functools.cache
def _seg_rxn_kernel():
    return pl.kernel(
        _seg_rxn_body,
        mesh=_mesh(),
        out_type=jax.ShapeDtypeStruct((NC * MSG * R_PAD,), jnp.float32),
        compiler_params=pltpu.CompilerParams(needs_layout_passes=False),
        scratch_types=[
            pltpu.VMEM((R_PAD,), jnp.float32),
            pltpu.VMEM((R_PAD,), jnp.float32),
            pltpu.VMEM((8, CW), jnp.int32),
            pltpu.VMEM((8, CW), jnp.int32),
            pltpu.VMEM((8, CW), jnp.float32),
            pltpu.VMEM((8, CW), jnp.float32),
            pltpu.VMEM((8, CW), jnp.float32),
            pltpu.VMEM((8, CW), jnp.float32),
            pltpu.SemaphoreType.DMA,
            pltpu.SemaphoreType.DMA,
        ],
    )


def _seg_rxn_body(msgt_hbm, rxn_hbm, zero_hbm, out_hbm,
                  acc0, acc1, idx0, idx1, v0a, v0b, v1a, v1b, sem0, sem1):
    c = lax.axis_index("c")
    s = lax.axis_index("s")
    p0 = 2 * s
    jbase = c * NJ_SC
    idxb = (idx0, idx1)
    vb0 = (v0a, v0b)
    vb1 = (v1a, v1b)
    sems = (sem0, sem1)
    pltpu.sync_copy(zero_hbm, acc0)
    pltpu.sync_copy(zero_hbm, acc1)

    def fire(g, m, b):
        r0 = jbase + g * 8
        m0 = m * CW
        return [
            pltpu.async_copy(rxn_hbm.at[pl.ds(r0, 8), pl.ds(m0, CW)],
                             idxb[b], sems[b]),
            pltpu.async_copy(msgt_hbm.at[p0, pl.ds(r0, 8), pl.ds(m0, CW)],
                             vb0[b], sems[b]),
            pltpu.async_copy(msgt_hbm.at[p0 + 1, pl.ds(r0, 8), pl.ds(m0, CW)],
                             vb1[b], sems[b]),
        ]

    def group(g, carry):
        hs = fire(g, 0, 0)
        for m in range(2048 // CW):
            b = m % 2
            for h in hs:
                h.wait()
            if m < 2048 // CW - 1:
                hs = fire(g, m + 1, 1 - b)

            nk = CW // 16

            def vec(t, cr):
                r = t // nk
                k16 = (t % nk) * 16
                iv = idxb[b][r, pl.ds(k16, 16)]
                plsc.addupdate_scatter(acc0, [iv],
                                       vb0[b][r, pl.ds(k16, 16)])
                plsc.addupdate_scatter(acc1, [iv],
                                       vb1[b][r, pl.ds(k16, 16)])
                return cr

            plsc.parallel_loop(0, 8 * nk, unroll=4, carry=jnp.int32(0))(vec)
        return carry

    lax.fori_loop(0, NJ_SC // 8, group, 0)
    obase = c * MSG * R_PAD + p0 * R_PAD
    pltpu.sync_copy(acc0, out_hbm.at[pl.ds(obase, R_PAD)])
    pltpu.sync_copy(acc1, out_hbm.at[pl.ds(obase + R_PAD, R_PAD)])


def _rate_body(ha_ref, hb_ref, w3t_ref, b3c_ref, w4_ref, b4_ref, logk_ref, out_ref):
    h = ha_ref[...] + hb_ref[...]
    t = jnp.tanh(
        jnp.dot(w3t_ref[...], h, preferred_element_type=jnp.float32)
        + b3c_ref[...]
    )
    rate = jnp.dot(w4_ref[...], t, preferred_element_type=jnp.float32) + b4_ref[0, 0]
    sp = jnp.maximum(rate, 0.0) + jnp.log(1.0 + jnp.exp(-jnp.abs(rate)))
    out_ref[...] = jnp.exp(logk_ref[...] * 2.302585092994046) * sp


def _rate(ha, hb, W3t, b3c, w4r, b4r, logk2):
    blk = 3584
    grid = R_PAD // blk
    return pl.pallas_call(
        _rate_body,
        grid=(grid,),
        in_specs=[
            pl.BlockSpec((MSG, blk), lambda i: (0, i)),
            pl.BlockSpec((MSG, blk), lambda i: (0, i)),
            pl.BlockSpec((HID, MSG), lambda i: (0, 0)),
            pl.BlockSpec((HID, 1), lambda i: (0, 0)),
            pl.BlockSpec((1, HID), lambda i: (0, 0)),
            pl.BlockSpec((1, 1), lambda i: (0, 0)),
            pl.BlockSpec((1, blk), lambda i: (0, i)),
        ],
        out_specs=pl.BlockSpec((1, blk), lambda i: (0, i)),
        out_shape=jax.ShapeDtypeStruct((1, R_PAD), jnp.float32),
    )(ha, hb, W3t, b3c, w4r, b4r, logk2)


@functools.cache
def _seg_met_kernel():
    return pl.kernel(
        _seg_met_body,
        mesh=_mesh(),
        out_type=jax.ShapeDtypeStruct((NW * R_PAD,), jnp.float32),
        compiler_params=pltpu.CompilerParams(needs_layout_passes=False),
        scratch_types=[
            pltpu.VMEM((R_PAD,), jnp.float32),
            pltpu.VMEM((R_PAD,), jnp.float32),
            pltpu.VMEM((8, 128), jnp.int32),
            pltpu.VMEM((8, 128), jnp.int32),
            pltpu.VMEM((8, 128), jnp.int32),
            pltpu.VMEM((8, 128), jnp.int32),
            pltpu.VMEM((8, 128), jnp.float32),
            pltpu.VMEM((8, 128), jnp.float32),
            pltpu.SemaphoreType.DMA,
            pltpu.SemaphoreType.DMA,
        ],
    )


def _seg_met_body(v_hbm, rxn_hbm, met_hbm, sto_hbm, zero_hbm, out_hbm,
                  acc, v_v, rxn0, rxn1, met0, met1, sto0, sto1, sem0, sem1):
    c = lax.axis_index("c")
    s = lax.axis_index("s")
    wid = s * NC + c
    rxnb = (rxn0, rxn1)
    metb = (met0, met1)
    stob = (sto0, sto1)
    sems = (sem0, sem1)
    pltpu.sync_copy(zero_hbm, acc)
    pltpu.sync_copy(v_hbm, v_v)
    rbase = wid * 392

    def fire(g, b):
        r0 = rbase + g * 8
        return [pltpu.async_copy(rxn_hbm.at[pl.ds(r0, 8)], rxnb[b], sems[b]),
                pltpu.async_copy(met_hbm.at[pl.ds(r0, 8)], metb[b], sems[b]),
                pltpu.async_copy(sto_hbm.at[pl.ds(r0, 8)], stob[b], sems[b])]

    hs = fire(0, 0)
    for g in range(49):
        b = g % 2
        for h in hs:
            h.wait()
        if g < 48:
            hs = fire(g + 1, 1 - b)

        def vec(t, cr, b=b):
            r = t // 8
            k16 = (t % 8) * 16
            rx = rxnb[b][r, pl.ds(k16, 16)]
            mt = metb[b][r, pl.ds(k16, 16)]
            st = stob[b][r, pl.ds(k16, 16)]
            vv = plsc.load_gather(v_v, [rx])
            plsc.addupdate_scatter(acc, [mt], st * vv)
            return cr

        plsc.parallel_loop(0, 64, unroll=4, carry=jnp.int32(0))(vec)

    pltpu.sync_copy(acc, out_hbm.at[pl.ds(wid * R_PAD, R_PAD)])


def _sumw_body(a_ref, o_ref):
    o_ref[...] = jnp.sum(a_ref[...], axis=0)


def _sumw(a):
    blk = 7168
    return pl.pallas_call(
        _sumw_body,
        grid=(R_PAD // blk,),
        in_specs=[pl.BlockSpec((NW, blk), lambda i: (0, i))],
        out_specs=pl.BlockSpec((blk,), lambda i: (i,)),
        out_shape=jax.ShapeDtypeStruct((R_PAD,), jnp.float32),
    )(a)


def kernel(x, met_sub, rxn_sub, sto_sub, met_all, rxn_all, sto_all,
           W1, b1, W2, b2, W3, b3, W4, b4, log_k):
    i32 = jnp.int32
    f32 = jnp.float32

    conc = jnp.pad(x[:, 3], (0, R_PAD - N_MET))

    ps = E_SUB_PAD - E_SUB
    met_sub_p = jnp.concatenate(
        [met_sub.astype(i32), jnp.arange(ps, dtype=i32) % N_MET])
    rxn_sub_p = jnp.concatenate(
        [rxn_sub.astype(i32), N_RXN + jnp.arange(ps, dtype=i32) % (R_PAD - N_RXN)])
    sto_sub_p = jnp.pad(sto_sub.astype(f32), (0, ps))

    pa = E_ALL_PAD - E_ALL
    met_all_p = jnp.concatenate(
        [met_all.astype(i32), jnp.arange(pa, dtype=i32) % N_MET])
    rxn_all_p = jnp.concatenate(
        [rxn_all.astype(i32), jnp.arange(pa, dtype=i32) % N_RXN])
    sto_all_p = jnp.pad(sto_all.astype(f32), (0, pa))

    c_sub = _gather_c_kernel()(conc, met_sub_p.reshape(-1, 128))

    in_t = jnp.stack([c_sub.reshape(-1), sto_sub_p], axis=0)
    msgt = _edge_mlp(in_t, W1.astype(f32).T, b1.reshape(HID, 1),
                     W2.astype(f32).T, b2.reshape(MSG, 1))

    h2 = _seg_rxn_kernel()(msgt, rxn_sub_p.reshape(NJ, 2048),
                           jnp.zeros((R_PAD,), dtype=f32))
    h2t = h2.reshape(NC, MSG, R_PAD)

    logk_p = jnp.pad(log_k.astype(f32), (0, R_PAD - N_RXN))
    v = _rate(h2t[0], h2t[1], W3.astype(f32).T, b3.reshape(HID, 1),
              W4.reshape(1, HID), b4.reshape(1, 1),
              logk_p.reshape(1, R_PAD)).reshape(R_PAD)

    dx32 = _seg_met_kernel()(v, rxn_all_p.reshape(-1, 128),
                             met_all_p.reshape(-1, 128),
                             sto_all_p.reshape(-1, 128),
                             jnp.zeros((R_PAD,), dtype=f32))

    dxdt = _sumw(dx32.reshape(NW, R_PAD))
    return dxdt[:N_MET, None]


# speedup vs baseline: 46.7535x; 1.0460x over previous
import functools

import jax
import jax.numpy as jnp
from jax import lax
from jax.experimental import pallas as pl
from jax.experimental.pallas import tpu as pltpu
from jax.experimental.pallas import tpu_sc as plsc

N_MET = 50000
N_RXN = 50000
E_SUB = 800000
E_ALL = 1600000
HID = 128
MSG = 32

NC = 2
NS = 16
NW = NC * NS

R_PAD = 50176
E_SUB_PAD = 819200
E_ALL_PAD = 1605632

@functools.cache
def _mesh():
    return plsc.VectorSubcoreMesh(core_axis_name="c", subcore_axis_name="s")


@functools.cache
def _gather_c_kernel():
    return pl.kernel(
        _gather_c_body,
        mesh=_mesh(),
        out_type=jax.ShapeDtypeStruct((E_SUB_PAD // 128, 128), jnp.float32),
        compiler_params=pltpu.CompilerParams(needs_layout_passes=False),
        scratch_types=[
            pltpu.VMEM((R_PAD,), jnp.float32),
            pltpu.VMEM((200, 128), jnp.int32),
            pltpu.VMEM((200, 128), jnp.float32),
        ],
    )


def _gather_c_body(conc_hbm, met_hbm, out_hbm, conc_v, idx_v, out_v):
    c = lax.axis_index("c")
    s = lax.axis_index("s")
    wid = s * NC + c
    base = wid * 200
    pltpu.sync_copy(conc_hbm, conc_v)
    pltpu.sync_copy(met_hbm.at[pl.ds(base, 200)], idx_v)

    def row(r, carry):
        for k in range(8):
            idx = idx_v[r, pl.ds(k * 16, 16)]
            out_v[r, pl.ds(k * 16, 16)] = plsc.load_gather(conc_v, [idx])
        return carry

    lax.fori_loop(0, 200, row, 0)
    pltpu.sync_copy(out_v, out_hbm.at[pl.ds(base, 200)])


def _edge_mlp_body(in_ref, w1t_ref, b1c_ref, w2t_ref, b2c_ref, out_ref):
    h = jnp.tanh(
        jnp.dot(w1t_ref[...], in_ref[...], preferred_element_type=jnp.float32)
        + b1c_ref[...]
    )
    o = jnp.dot(w2t_ref[...], h, preferred_element_type=jnp.float32) + b2c_ref[...]
    out_ref[...] = o.reshape(out_ref.shape)


def _edge_mlp(in_t, W1t, b1c, W2t, b2c):
    blkj = 8
    blk = blkj * 2048
    grid = E_SUB_PAD // blk
    return pl.pallas_call(
        _edge_mlp_body,
        grid=(grid,),
        in_specs=[
            pl.BlockSpec((2, blk), lambda i: (0, i)),
            pl.BlockSpec((HID, 2), lambda i: (0, 0)),
            pl.BlockSpec((HID, 1), lambda i: (0, 0)),
            pl.BlockSpec((MSG, HID), lambda i: (0, 0)),
            pl.BlockSpec((MSG, 1), lambda i: (0, 0)),
        ],
        out_specs=pl.BlockSpec((MSG, blkj, 2048), lambda i: (0, i, 0)),
        out_shape=jax.ShapeDtypeStruct((MSG, E_SUB_PAD // 2048, 2048), jnp.float32),
    )(in_t, W1t, b1c, W2t, b2c)


NJ = E_SUB_PAD // 2048
NJ_SC = NJ // NC
CW = 512


@functools.cache
def _seg_rxn_kernel():
    return pl.kernel(
        _seg_rxn_body,
        mesh=_mesh(),
        out_type=jax.ShapeDtypeStruct((NC * MSG * R_PAD,), jnp.float32),
        compiler_params=pltpu.CompilerParams(needs_layout_passes=False),
        scratch_types=[
            pltpu.VMEM((R_PAD,), jnp.float32),
            pltpu.VMEM((R_PAD,), jnp.float32),
            pltpu.VMEM((8, CW), jnp.int32),
            pltpu.VMEM((8, CW), jnp.int32),
            pltpu.VMEM((8, CW), jnp.float32),
            pltpu.VMEM((8, CW), jnp.float32),
            pltpu.VMEM((8, CW), jnp.float32),
            pltpu.VMEM((8, CW), jnp.float32),
            pltpu.SemaphoreType.DMA,
            pltpu.SemaphoreType.DMA,
        ],
    )


def _seg_rxn_body(msgt_hbm, rxn_hbm, out_hbm,
                  acc0, acc1, idx0, idx1, v0a, v0b, v1a, v1b, sem0, sem1):
    c = lax.axis_index("c")
    s = lax.axis_index("s")
    p0 = 2 * s
    jbase = c * NJ_SC
    idxb = (idx0, idx1)
    vb0 = (v0a, v0b)
    vb1 = (v1a, v1b)
    sems = (sem0, sem1)

    zv = jnp.zeros((16,), jnp.float32)

    def zr(t, cr):
        acc0[pl.ds(t * 16, 16)] = zv
        acc1[pl.ds(t * 16, 16)] = zv
        return cr

    plsc.parallel_loop(0, R_PAD // 16, unroll=4, carry=jnp.int32(0))(zr)

    def fire(g, m, b):
        r0 = jbase + g * 8
        m0 = m * CW
        return [
            pltpu.async_copy(rxn_hbm.at[pl.ds(r0, 8), pl.ds(m0, CW)],
                             idxb[b], sems[b]),
            pltpu.async_copy(msgt_hbm.at[p0, pl.ds(r0, 8), pl.ds(m0, CW)],
                             vb0[b], sems[b]),
            pltpu.async_copy(msgt_hbm.at[p0 + 1, pl.ds(r0, 8), pl.ds(m0, CW)],
                             vb1[b], sems[b]),
        ]

    def group(g, carry):
        hs = fire(g, 0, 0)
        for m in range(2048 // CW):
            b = m % 2
            for h in hs:
                h.wait()
            if m < 2048 // CW - 1:
                hs = fire(g, m + 1, 1 - b)

            nk = CW // 16

            def vec(t, cr):
                r = t // nk
                k16 = (t % nk) * 16
                iv = idxb[b][r, pl.ds(k16, 16)]
                plsc.addupdate_scatter(acc0, [iv],
                                       vb0[b][r, pl.ds(k16, 16)])
                plsc.addupdate_scatter(acc1, [iv],
                                       vb1[b][r, pl.ds(k16, 16)])
                return cr

            plsc.parallel_loop(0, 8 * nk, unroll=4, carry=jnp.int32(0))(vec)
        return carry

    lax.fori_loop(0, NJ_SC // 8, group, 0)
    obase = c * MSG * R_PAD + p0 * R_PAD
    pltpu.sync_copy(acc0, out_hbm.at[pl.ds(obase, R_PAD)])
    pltpu.sync_copy(acc1, out_hbm.at[pl.ds(obase + R_PAD, R_PAD)])


def _rate_body(ha_ref, hb_ref, w3t_ref, b3c_ref, w4_ref, b4_ref, logk_ref, out_ref):
    h = ha_ref[...] + hb_ref[...]
    t = jnp.tanh(
        jnp.dot(w3t_ref[...], h, preferred_element_type=jnp.float32)
        + b3c_ref[...]
    )
    rate = jnp.dot(w4_ref[...], t, preferred_element_type=jnp.float32) + b4_ref[0, 0]
    sp = jnp.maximum(rate, 0.0) + jnp.log(1.0 + jnp.exp(-jnp.abs(rate)))
    out_ref[...] = jnp.exp(logk_ref[...] * 2.302585092994046) * sp


def _rate(ha, hb, W3t, b3c, w4r, b4r, logk2):
    blk = 3584
    grid = R_PAD // blk
    return pl.pallas_call(
        _rate_body,
        grid=(grid,),
        in_specs=[
            pl.BlockSpec((MSG, blk), lambda i: (0, i)),
            pl.BlockSpec((MSG, blk), lambda i: (0, i)),
            pl.BlockSpec((HID, MSG), lambda i: (0, 0)),
            pl.BlockSpec((HID, 1), lambda i: (0, 0)),
            pl.BlockSpec((1, HID), lambda i: (0, 0)),
            pl.BlockSpec((1, 1), lambda i: (0, 0)),
            pl.BlockSpec((1, blk), lambda i: (0, i)),
        ],
        out_specs=pl.BlockSpec((1, blk), lambda i: (0, i)),
        out_shape=jax.ShapeDtypeStruct((1, R_PAD), jnp.float32),
    )(ha, hb, W3t, b3c, w4r, b4r, logk2)


@functools.cache
def _seg_met_kernel():
    return pl.kernel(
        _seg_met_body,
        mesh=_mesh(),
        out_type=jax.ShapeDtypeStruct((NW * R_PAD,), jnp.float32),
        compiler_params=pltpu.CompilerParams(needs_layout_passes=False),
        scratch_types=[
            pltpu.VMEM((R_PAD,), jnp.float32),
            pltpu.VMEM((R_PAD,), jnp.float32),
            pltpu.VMEM((8, 128), jnp.int32),
            pltpu.VMEM((8, 128), jnp.int32),
            pltpu.VMEM((8, 128), jnp.int32),
            pltpu.VMEM((8, 128), jnp.int32),
            pltpu.VMEM((8, 128), jnp.float32),
            pltpu.VMEM((8, 128), jnp.float32),
            pltpu.SemaphoreType.DMA,
            pltpu.SemaphoreType.DMA,
        ],
    )


def _seg_met_body(v_hbm, rxn_hbm, met_hbm, sto_hbm, out_hbm,
                  acc, v_v, rxn0, rxn1, met0, met1, sto0, sto1, sem0, sem1):
    c = lax.axis_index("c")
    s = lax.axis_index("s")
    wid = s * NC + c
    rxnb = (rxn0, rxn1)
    metb = (met0, met1)
    stob = (sto0, sto1)
    sems = (sem0, sem1)
    vh = pltpu.async_copy(v_hbm, v_v, sems[0])

    zv = jnp.zeros((16,), jnp.float32)

    def zr(t, cr):
        acc[pl.ds(t * 16, 16)] = zv
        return cr

    plsc.parallel_loop(0, R_PAD // 16, unroll=4, carry=jnp.int32(0))(zr)
    vh.wait()
    rbase = wid * 392

    def fire(g, b):
        r0 = rbase + g * 8
        return [pltpu.async_copy(rxn_hbm.at[pl.ds(r0, 8)], rxnb[b], sems[b]),
                pltpu.async_copy(met_hbm.at[pl.ds(r0, 8)], metb[b], sems[b]),
                pltpu.async_copy(sto_hbm.at[pl.ds(r0, 8)], stob[b], sems[b])]

    hs = fire(0, 0)
    for g in range(49):
        b = g % 2
        for h in hs:
            h.wait()
        if g < 48:
            hs = fire(g + 1, 1 - b)

        def vec(t, cr, b=b):
            r = t // 8
            k16 = (t % 8) * 16
            rx = rxnb[b][r, pl.ds(k16, 16)]
            mt = metb[b][r, pl.ds(k16, 16)]
            st = stob[b][r, pl.ds(k16, 16)]
            vv = plsc.load_gather(v_v, [rx])
            plsc.addupdate_scatter(acc, [mt], st * vv)
            return cr

        plsc.parallel_loop(0, 64, unroll=4, carry=jnp.int32(0))(vec)

    pltpu.sync_copy(acc, out_hbm.at[pl.ds(wid * R_PAD, R_PAD)])


def _sumw_body(a_ref, o_ref):
    o_ref[...] = jnp.sum(a_ref[...], axis=0)


def _sumw(a):
    blk = 7168
    return pl.pallas_call(
        _sumw_body,
        grid=(R_PAD // blk,),
        in_specs=[pl.BlockSpec((NW, blk), lambda i: (0, i))],
        out_specs=pl.BlockSpec((blk,), lambda i: (i,)),
        out_shape=jax.ShapeDtypeStruct((R_PAD,), jnp.float32),
    )(a)


def kernel(x, met_sub, rxn_sub, sto_sub, met_all, rxn_all, sto_all,
           W1, b1, W2, b2, W3, b3, W4, b4, log_k):
    i32 = jnp.int32
    f32 = jnp.float32

    conc = jnp.pad(x[:, 3], (0, R_PAD - N_MET))

    ps = E_SUB_PAD - E_SUB
    met_sub_p = jnp.concatenate(
        [met_sub.astype(i32), jnp.arange(ps, dtype=i32) % N_MET])
    rxn_sub_p = jnp.concatenate(
        [rxn_sub.astype(i32), N_RXN + jnp.arange(ps, dtype=i32) % (R_PAD - N_RXN)])
    sto_sub_p = jnp.pad(sto_sub.astype(f32), (0, ps))

    pa = E_ALL_PAD - E_ALL
    met_all_p = jnp.concatenate(
        [met_all.astype(i32), jnp.arange(pa, dtype=i32) % N_MET])
    rxn_all_p = jnp.concatenate(
        [rxn_all.astype(i32), jnp.arange(pa, dtype=i32) % N_RXN])
    sto_all_p = jnp.pad(sto_all.astype(f32), (0, pa))

    c_sub = _gather_c_kernel()(conc, met_sub_p.reshape(-1, 128))

    in_t = jnp.stack([c_sub.reshape(-1), sto_sub_p], axis=0)
    msgt = _edge_mlp(in_t, W1.astype(f32).T, b1.reshape(HID, 1),
                     W2.astype(f32).T, b2.reshape(MSG, 1))

    h2 = _seg_rxn_kernel()(msgt, rxn_sub_p.reshape(NJ, 2048))
    h2t = h2.reshape(NC, MSG, R_PAD)

    logk_p = jnp.pad(log_k.astype(f32), (0, R_PAD - N_RXN))
    v = _rate(h2t[0], h2t[1], W3.astype(f32).T, b3.reshape(HID, 1),
              W4.reshape(1, HID), b4.reshape(1, 1),
              logk_p.reshape(1, R_PAD)).reshape(R_PAD)

    dx32 = _seg_met_kernel()(v, rxn_all_p.reshape(-1, 128),
                             met_all_p.reshape(-1, 128),
                             sto_all_p.reshape(-1, 128))

    dxdt = _sumw(dx32.reshape(NW, R_PAD))
    return dxdt[:N_MET, None]


# speedup vs baseline: 46.8082x; 1.0012x over previous
import functools

import jax
import jax.numpy as jnp
from jax import lax
from jax.experimental import pallas as pl
from jax.experimental.pallas import tpu as pltpu
from jax.experimental.pallas import tpu_sc as plsc

N_MET = 50000
N_RXN = 50000
E_SUB = 800000
E_ALL = 1600000
HID = 128
MSG = 32

NC = 2
NS = 16
NW = NC * NS

R_PAD = 50176
E_SUB_PAD = 819200
E_ALL_PAD = 1605632

@functools.cache
def _mesh():
    return plsc.VectorSubcoreMesh(core_axis_name="c", subcore_axis_name="s")


@functools.cache
def _gather_c_kernel():
    return pl.kernel(
        _gather_c_body,
        mesh=_mesh(),
        out_type=jax.ShapeDtypeStruct((E_SUB_PAD // 128, 128), jnp.float32),
        compiler_params=pltpu.CompilerParams(needs_layout_passes=False),
        scratch_types=[
            pltpu.VMEM((R_PAD,), jnp.float32),
            pltpu.VMEM((200, 128), jnp.int32),
            pltpu.VMEM((200, 128), jnp.float32),
        ],
    )


def _gather_c_body(conc_hbm, met_hbm, out_hbm, conc_v, idx_v, out_v):
    c = lax.axis_index("c")
    s = lax.axis_index("s")
    wid = s * NC + c
    base = wid * 200
    pltpu.sync_copy(conc_hbm, conc_v)
    pltpu.sync_copy(met_hbm.at[pl.ds(base, 200)], idx_v)

    def row(r, carry):
        for k in range(8):
            idx = idx_v[r, pl.ds(k * 16, 16)]
            out_v[r, pl.ds(k * 16, 16)] = plsc.load_gather(conc_v, [idx])
        return carry

    lax.fori_loop(0, 200, row, 0)
    pltpu.sync_copy(out_v, out_hbm.at[pl.ds(base, 200)])


def _edge_mlp_body(in_ref, w1t_ref, b1c_ref, w2t_ref, b2c_ref, out_ref):
    h = jnp.tanh(
        jnp.dot(w1t_ref[...], in_ref[...], preferred_element_type=jnp.float32)
        + b1c_ref[...]
    )
    o = jnp.dot(w2t_ref[...], h, preferred_element_type=jnp.float32) + b2c_ref[...]
    out_ref[...] = o.reshape(out_ref.shape)


def _edge_mlp(in_t, W1t, b1c, W2t, b2c):
    blkj = 16
    blk = blkj * 2048
    grid = E_SUB_PAD // blk
    return pl.pallas_call(
        _edge_mlp_body,
        grid=(grid,),
        in_specs=[
            pl.BlockSpec((2, blk), lambda i: (0, i)),
            pl.BlockSpec((HID, 2), lambda i: (0, 0)),
            pl.BlockSpec((HID, 1), lambda i: (0, 0)),
            pl.BlockSpec((MSG, HID), lambda i: (0, 0)),
            pl.BlockSpec((MSG, 1), lambda i: (0, 0)),
        ],
        out_specs=pl.BlockSpec((MSG, blkj, 2048), lambda i: (0, i, 0)),
        out_shape=jax.ShapeDtypeStruct((MSG, E_SUB_PAD // 2048, 2048), jnp.float32),
    )(in_t, W1t, b1c, W2t, b2c)


NJ = E_SUB_PAD // 2048
NJ_SC = NJ // NC
CW = 512


@functools.cache
def _seg_rxn_kernel():
    return pl.kernel(
        _seg_rxn_body,
        mesh=_mesh(),
        out_type=jax.ShapeDtypeStruct((NC * MSG * R_PAD,), jnp.float32),
        compiler_params=pltpu.CompilerParams(needs_layout_passes=False),
        scratch_types=[
            pltpu.VMEM((R_PAD,), jnp.float32),
            pltpu.VMEM((R_PAD,), jnp.float32),
            pltpu.VMEM((8, CW), jnp.int32),
            pltpu.VMEM((8, CW), jnp.int32),
            pltpu.VMEM((8, CW), jnp.float32),
            pltpu.VMEM((8, CW), jnp.float32),
            pltpu.VMEM((8, CW), jnp.float32),
            pltpu.VMEM((8, CW), jnp.float32),
            pltpu.SemaphoreType.DMA,
            pltpu.SemaphoreType.DMA,
        ],
    )


def _seg_rxn_body(msgt_hbm, rxn_hbm, out_hbm,
                  acc0, acc1, idx0, idx1, v0a, v0b, v1a, v1b, sem0, sem1):
    c = lax.axis_index("c")
    s = lax.axis_index("s")
    p0 = 2 * s
    jbase = c * NJ_SC
    idxb = (idx0, idx1)
    vb0 = (v0a, v0b)
    vb1 = (v1a, v1b)
    sems = (sem0, sem1)

    zv = jnp.zeros((16,), jnp.float32)

    def zr(t, cr):
        acc0[pl.ds(t * 16, 16)] = zv
        acc1[pl.ds(t * 16, 16)] = zv
        return cr

    plsc.parallel_loop(0, R_PAD // 16, unroll=4, carry=jnp.int32(0))(zr)

    def fire(g, m, b):
        r0 = jbase + g * 8
        m0 = m * CW
        return [
            pltpu.async_copy(rxn_hbm.at[pl.ds(r0, 8), pl.ds(m0, CW)],
                             idxb[b], sems[b]),
            pltpu.async_copy(msgt_hbm.at[p0, pl.ds(r0, 8), pl.ds(m0, CW)],
                             vb0[b], sems[b]),
            pltpu.async_copy(msgt_hbm.at[p0 + 1, pl.ds(r0, 8), pl.ds(m0, CW)],
                             vb1[b], sems[b]),
        ]

    def group(g, carry):
        hs = fire(g, 0, 0)
        for m in range(2048 // CW):
            b = m % 2
            for h in hs:
                h.wait()
            if m < 2048 // CW - 1:
                hs = fire(g, m + 1, 1 - b)

            nk = CW // 16

            def vec(t, cr):
                r = t // nk
                k16 = (t % nk) * 16
                iv = idxb[b][r, pl.ds(k16, 16)]
                plsc.addupdate_scatter(acc0, [iv],
                                       vb0[b][r, pl.ds(k16, 16)])
                plsc.addupdate_scatter(acc1, [iv],
                                       vb1[b][r, pl.ds(k16, 16)])
                return cr

            plsc.parallel_loop(0, 8 * nk, unroll=8, carry=jnp.int32(0))(vec)
        return carry

    lax.fori_loop(0, NJ_SC // 8, group, 0)
    obase = c * MSG * R_PAD + p0 * R_PAD
    pltpu.sync_copy(acc0, out_hbm.at[pl.ds(obase, R_PAD)])
    pltpu.sync_copy(acc1, out_hbm.at[pl.ds(obase + R_PAD, R_PAD)])


def _rate_body(ha_ref, hb_ref, w3t_ref, b3c_ref, w4_ref, b4_ref, logk_ref, out_ref):
    h = ha_ref[...] + hb_ref[...]
    t = jnp.tanh(
        jnp.dot(w3t_ref[...], h, preferred_element_type=jnp.float32)
        + b3c_ref[...]
    )
    rate = jnp.dot(w4_ref[...], t, preferred_element_type=jnp.float32) + b4_ref[0, 0]
    sp = jnp.maximum(rate, 0.0) + jnp.log(1.0 + jnp.exp(-jnp.abs(rate)))
    out_ref[...] = jnp.exp(logk_ref[...] * 2.302585092994046) * sp


def _rate(ha, hb, W3t, b3c, w4r, b4r, logk2):
    blk = 3584
    grid = R_PAD // blk
    return pl.pallas_call(
        _rate_body,
        grid=(grid,),
        in_specs=[
            pl.BlockSpec((MSG, blk), lambda i: (0, i)),
            pl.BlockSpec((MSG, blk), lambda i: (0, i)),
            pl.BlockSpec((HID, MSG), lambda i: (0, 0)),
            pl.BlockSpec((HID, 1), lambda i: (0, 0)),
            pl.BlockSpec((1, HID), lambda i: (0, 0)),
            pl.BlockSpec((1, 1), lambda i: (0, 0)),
            pl.BlockSpec((1, blk), lambda i: (0, i)),
        ],
        out_specs=pl.BlockSpec((1, blk), lambda i: (0, i)),
        out_shape=jax.ShapeDtypeStruct((1, R_PAD), jnp.float32),
    )(ha, hb, W3t, b3c, w4r, b4r, logk2)


@functools.cache
def _seg_met_kernel():
    return pl.kernel(
        _seg_met_body,
        mesh=_mesh(),
        out_type=jax.ShapeDtypeStruct((NW * R_PAD,), jnp.float32),
        compiler_params=pltpu.CompilerParams(needs_layout_passes=False),
        scratch_types=[
            pltpu.VMEM((R_PAD,), jnp.float32),
            pltpu.VMEM((R_PAD,), jnp.float32),
            pltpu.VMEM((8, 128), jnp.int32),
            pltpu.VMEM((8, 128), jnp.int32),
            pltpu.VMEM((8, 128), jnp.int32),
            pltpu.VMEM((8, 128), jnp.int32),
            pltpu.VMEM((8, 128), jnp.float32),
            pltpu.VMEM((8, 128), jnp.float32),
            pltpu.SemaphoreType.DMA,
            pltpu.SemaphoreType.DMA,
        ],
    )


def _seg_met_body(v_hbm, rxn_hbm, met_hbm, sto_hbm, out_hbm,
                  acc, v_v, rxn0, rxn1, met0, met1, sto0, sto1, sem0, sem1):
    c = lax.axis_index("c")
    s = lax.axis_index("s")
    wid = s * NC + c
    rxnb = (rxn0, rxn1)
    metb = (met0, met1)
    stob = (sto0, sto1)
    sems = (sem0, sem1)
    vh = pltpu.async_copy(v_hbm, v_v, sems[0])

    zv = jnp.zeros((16,), jnp.float32)

    def zr(t, cr):
        acc[pl.ds(t * 16, 16)] = zv
        return cr

    plsc.parallel_loop(0, R_PAD // 16, unroll=4, carry=jnp.int32(0))(zr)
    vh.wait()
    rbase = wid * 392

    def fire(g, b):
        r0 = rbase + g * 8
        return [pltpu.async_copy(rxn_hbm.at[pl.ds(r0, 8)], rxnb[b], sems[b]),
                pltpu.async_copy(met_hbm.at[pl.ds(r0, 8)], metb[b], sems[b]),
                pltpu.async_copy(sto_hbm.at[pl.ds(r0, 8)], stob[b], sems[b])]

    hs = fire(0, 0)
    for g in range(49):
        b = g % 2
        for h in hs:
            h.wait()
        if g < 48:
            hs = fire(g + 1, 1 - b)

        def vec(t, cr, b=b):
            r = t // 8
            k16 = (t % 8) * 16
            rx = rxnb[b][r, pl.ds(k16, 16)]
            mt = metb[b][r, pl.ds(k16, 16)]
            st = stob[b][r, pl.ds(k16, 16)]
            vv = plsc.load_gather(v_v, [rx])
            plsc.addupdate_scatter(acc, [mt], st * vv)
            return cr

        plsc.parallel_loop(0, 64, unroll=8, carry=jnp.int32(0))(vec)

    pltpu.sync_copy(acc, out_hbm.at[pl.ds(wid * R_PAD, R_PAD)])


def _sumw_body(a_ref, o_ref):
    o_ref[...] = jnp.sum(a_ref[...], axis=0)


def _sumw(a):
    blk = 7168
    return pl.pallas_call(
        _sumw_body,
        grid=(R_PAD // blk,),
        in_specs=[pl.BlockSpec((NW, blk), lambda i: (0, i))],
        out_specs=pl.BlockSpec((blk,), lambda i: (i,)),
        out_shape=jax.ShapeDtypeStruct((R_PAD,), jnp.float32),
    )(a)


def kernel(x, met_sub, rxn_sub, sto_sub, met_all, rxn_all, sto_all,
           W1, b1, W2, b2, W3, b3, W4, b4, log_k):
    i32 = jnp.int32
    f32 = jnp.float32

    conc = jnp.pad(x[:, 3], (0, R_PAD - N_MET))

    ps = E_SUB_PAD - E_SUB
    met_sub_p = jnp.concatenate(
        [met_sub.astype(i32), jnp.arange(ps, dtype=i32) % N_MET])
    rxn_sub_p = jnp.concatenate(
        [rxn_sub.astype(i32), N_RXN + jnp.arange(ps, dtype=i32) % (R_PAD - N_RXN)])
    sto_sub_p = jnp.pad(sto_sub.astype(f32), (0, ps))

    pa = E_ALL_PAD - E_ALL
    met_all_p = jnp.concatenate(
        [met_all.astype(i32), jnp.arange(pa, dtype=i32) % N_MET])
    rxn_all_p = jnp.concatenate(
        [rxn_all.astype(i32), jnp.arange(pa, dtype=i32) % N_RXN])
    sto_all_p = jnp.pad(sto_all.astype(f32), (0, pa))

    c_sub = _gather_c_kernel()(conc, met_sub_p.reshape(-1, 128))

    in_t = jnp.stack([c_sub.reshape(-1), sto_sub_p], axis=0)
    msgt = _edge_mlp(in_t, W1.astype(f32).T, b1.reshape(HID, 1),
                     W2.astype(f32).T, b2.reshape(MSG, 1))

    h2 = _seg_rxn_kernel()(msgt, rxn_sub_p.reshape(NJ, 2048))
    h2t = h2.reshape(NC, MSG, R_PAD)

    logk_p = jnp.pad(log_k.astype(f32), (0, R_PAD - N_RXN))
    v = _rate(h2t[0], h2t[1], W3.astype(f32).T, b3.reshape(HID, 1),
              W4.reshape(1, HID), b4.reshape(1, 1),
              logk_p.reshape(1, R_PAD)).reshape(R_PAD)

    dx32 = _seg_met_kernel()(v, rxn_all_p.reshape(-1, 128),
                             met_all_p.reshape(-1, 128),
                             sto_all_p.reshape(-1, 128))

    dxdt = _sumw(dx32.reshape(NW, R_PAD))
    return dxdt[:N_MET, None]
